# Initial kernel scaffold; baseline (speedup 1.0000x reference)
import jax
import jax.numpy as jnp
from jax.experimental import pallas as pl


def kernel(x, edge_index_list, Wq1, bq1, Wk1, bk1, Wv1, bv1, Ws1, bs1, Wq2, bq2, Wk2, bk2, Wv2, bv2, Ws2, bs2):
    raise NotImplementedError("write your pallas kernel here")


# speedup vs baseline: 9.1077x; 9.1077x over previous
import dataclasses
import functools

import jax
import jax.numpy as jnp
from jax import lax
from jax.experimental import pallas as pl
from jax.experimental.pallas import tpu as pltpu
from jax.experimental.pallas import tpu_sc as plsc

F32 = jnp.float32
I32 = jnp.int32
LN = 16
H = 2
CH = 256
QW = 64
NQ = CH // QW
W = 128
NSUB = 16


def _qkv_body(x_ref, wq_ref, bq_ref, wk_ref, bk_ref, wv_ref, bv_ref,
              ws_ref, bs_ref, q_ref, k_ref, v_ref, s_ref, *, bn, n_valid):
    i = pl.program_id(0)
    rows = i * bn + lax.broadcasted_iota(I32, (bn, 1), 0)
    m = (rows < n_valid).astype(F32)
    x = x_ref[...] * m
    q = (jnp.dot(x, wq_ref[...], preferred_element_type=F32) + bq_ref[...]) * m
    k = (jnp.dot(x, wk_ref[...], preferred_element_type=F32) + bk_ref[...]) * m
    v = (jnp.dot(x, wv_ref[...], preferred_element_type=F32) + bv_ref[...]) * m
    s = (jnp.dot(x, ws_ref[...], preferred_element_type=F32) + bs_ref[...]) * m
    q_ref[0], q_ref[1] = q[:, :CH], q[:, CH:]
    k_ref[0], k_ref[1] = k[:, :CH], k[:, CH:]
    for qq in range(H * NQ):
        v_ref[qq] = v[:, qq * QW:(qq + 1) * QW]
    s_ref[...] = s


@functools.lru_cache(maxsize=None)
def _qkv_call(np_, cin, n_valid, bn):
    return pl.pallas_call(
        functools.partial(_qkv_body, bn=bn, n_valid=n_valid),
        grid=(np_ // bn,),
        in_specs=[
            pl.BlockSpec((bn, cin), lambda i: (i, 0)),
            pl.BlockSpec((cin, H * CH), lambda i: (0, 0)),
            pl.BlockSpec((1, H * CH), lambda i: (0, 0)),
            pl.BlockSpec((cin, H * CH), lambda i: (0, 0)),
            pl.BlockSpec((1, H * CH), lambda i: (0, 0)),
            pl.BlockSpec((cin, H * CH), lambda i: (0, 0)),
            pl.BlockSpec((1, H * CH), lambda i: (0, 0)),
            pl.BlockSpec((cin, CH), lambda i: (0, 0)),
            pl.BlockSpec((1, CH), lambda i: (0, 0)),
        ],
        out_specs=[
            pl.BlockSpec((H, bn, CH), lambda i: (0, i, 0)),
            pl.BlockSpec((H, bn, CH), lambda i: (0, i, 0)),
            pl.BlockSpec((H * NQ, bn, QW), lambda i: (0, i, 0)),
            pl.BlockSpec((bn, CH), lambda i: (i, 0)),
        ],
        out_shape=[
            jax.ShapeDtypeStruct((H, np_, CH), F32),
            jax.ShapeDtypeStruct((H, np_, CH), F32),
            jax.ShapeDtypeStruct((H * NQ, np_, QW), F32),
            jax.ShapeDtypeStruct((np_, CH), F32),
        ],
    )


def _sc_body(qtab, ktab, vtab, dst_hbm, src_hbm, den_out, msg_out,
             dstb, srcb, idxq, idxs, qbuf, kbuf, vbuf, exstripe, albuf,
             zbuf, semq, semk, acc, *, np_, s_per, nb):
    c = lax.axis_index("c")
    t = lax.axis_index("s")
    rpt = np_ // NSUB
    cnp = c * np_
    lane = lax.iota(I32, LN)
    m_last = lane == (LN - 1)
    zeros = jnp.zeros((LN,), F32)
    zrows = zbuf.shape[0]

    @pl.loop(0, zrows)
    def _(r):
        for j in range(QW // LN):
            zbuf[r, pl.ds(j * LN, LN)] = zeros

    @pl.loop(0, rpt, step=zrows)
    def _(r):
        pltpu.sync_copy(zbuf, acc.at[pl.ds(t * rpt + r, zrows)])

    plsc.subcore_barrier()

    ebase0 = t * s_per

    @pl.loop(0, nb)
    def _(b):
        base = ebase0 + b * W
        pltpu.sync_copy(dst_hbm.at[pl.ds(base, W)], dstb)
        pltpu.sync_copy(src_hbm.at[pl.ds(base, W)], srcb)
        for j in range(W // LN):
            sl = pl.ds(j * LN, LN)
            idxq[sl] = dstb[sl] + cnp
            idxs[sl] = srcb[sl] + cnp
        cpq = pltpu.async_copy(qtab.at[idxq], qbuf, semq)
        cpk = pltpu.async_copy(ktab.at[idxs], kbuf, semk)
        cpq.wait()
        cpk.wait()

        @pl.loop(0, W)
        def _(e):
            a = qbuf[e, pl.ds(0, LN)] * kbuf[e, pl.ds(0, LN)]
            for j in range(1, CH // LN):
                sl = pl.ds(j * LN, LN)
                a = a + qbuf[e, sl] * kbuf[e, sl]
            plsc.store_scatter(albuf, [jnp.full((LN,), e, I32)],
                               plsc.cumsum(a), mask=m_last)

        eb = b * W
        for j in range(W // LN):
            ex = jnp.exp(albuf[pl.ds(j * LN, LN)] * (1.0 / 16.0))
            exstripe[pl.ds(eb + j * LN, LN)] = ex

    @pl.loop(0, nb)
    def _(b):
        base = ebase0 + b * W
        pltpu.sync_copy(dst_hbm.at[pl.ds(base, W)], dstb)
        eb = b * W

        @pl.loop(0, W)
        def _(e):
            wv = plsc.load_gather(exstripe, [jnp.full((LN,), eb + e, I32)])
            for j in range(QW // LN):
                vbuf[e, pl.ds(j * LN, LN)] = wv

        pltpu.sync_copy(vbuf, acc.at[dstb], add=True)

    plsc.subcore_barrier()
    pltpu.sync_copy(acc.at[pl.ds(t * rpt, rpt)],
                    den_out.at[pl.ds(cnp + t * rpt, rpt)])

    for quarter in range(NQ):
        voff = (NQ * c + quarter) * np_

        @pl.loop(0, rpt, step=zrows)
        def _(r):
            pltpu.sync_copy(zbuf, acc.at[pl.ds(t * rpt + r, zrows)])

        plsc.subcore_barrier()

        @pl.loop(0, nb)
        def _(b):
            base = ebase0 + b * W
            pltpu.sync_copy(dst_hbm.at[pl.ds(base, W)], dstb)
            pltpu.sync_copy(src_hbm.at[pl.ds(base, W)], srcb)
            for j in range(W // LN):
                sl = pl.ds(j * LN, LN)
                idxs[sl] = srcb[sl] + voff
            pltpu.async_copy(vtab.at[idxs], vbuf, semq).wait()
            eb = b * W

            @pl.loop(0, W)
            def _(e):
                wv = plsc.load_gather(exstripe, [jnp.full((LN,), eb + e, I32)])
                for j in range(QW // LN):
                    sl = pl.ds(j * LN, LN)
                    vbuf[e, sl] = vbuf[e, sl] * wv

            pltpu.sync_copy(vbuf, acc.at[dstb], add=True)

        plsc.subcore_barrier()
        pltpu.sync_copy(acc.at[pl.ds(t * rpt, rpt)],
                        msg_out.at[pl.ds(voff + t * rpt, rpt)])
        plsc.subcore_barrier()


@functools.lru_cache(maxsize=None)
def _sc_call(np_, s_per, nb):
    mesh = plsc.VectorSubcoreMesh(core_axis_name="c", subcore_axis_name="s")
    cp = pltpu.CompilerParams()
    for fld, val in (("needs_layout_passes", False),
                     ("use_tc_tiling_on_sc", False)):
        if fld in pltpu.CompilerParams.__dataclass_fields__:
            cp = dataclasses.replace(cp, **{fld: val})
    return pl.kernel(
        functools.partial(_sc_body, np_=np_, s_per=s_per, nb=nb),
        out_type=(
            jax.ShapeDtypeStruct((H * np_, QW), F32),
            jax.ShapeDtypeStruct((H * NQ * np_, QW), F32),
        ),
        mesh=mesh,
        compiler_params=cp,
        scratch_types=[
            pltpu.VMEM((W,), I32),
            pltpu.VMEM((W,), I32),
            pltpu.VMEM((W,), I32),
            pltpu.VMEM((W,), I32),
            pltpu.VMEM((W, CH), F32),
            pltpu.VMEM((W, CH), F32),
            pltpu.VMEM((W, QW), F32),
            pltpu.VMEM((s_per,), F32),
            pltpu.VMEM((W,), F32),
            pltpu.VMEM((32, QW), F32),
            pltpu.SemaphoreType.DMA,
            pltpu.SemaphoreType.DMA,
            pltpu.VMEM_SHARED((np_, QW), F32),
        ],
    )


def _fin_body(msg_ref, den_ref, s_ref, o_ref, *, act):
    o = s_ref[...]
    ms = []
    for h in range(H):
        d = den_ref[h, :, 0:1] + 1e-16
        m = jnp.concatenate([msg_ref[h, qq] for qq in range(NQ)], axis=1) / d
        ms.append(m)
    o = o + sum(ms) * (1.0 / H)
    if act:
        o = jnp.where(o >= 0, o, 0.1 * o)
    o_ref[...] = o


@functools.lru_cache(maxsize=None)
def _fin_call(np_, bn, act):
    return pl.pallas_call(
        functools.partial(_fin_body, act=act),
        grid=(np_ // bn,),
        in_specs=[
            pl.BlockSpec((H, NQ, bn, QW), lambda i: (0, 0, i, 0)),
            pl.BlockSpec((H, bn, QW), lambda i: (0, i, 0)),
            pl.BlockSpec((bn, CH), lambda i: (i, 0)),
        ],
        out_specs=pl.BlockSpec((bn, CH), lambda i: (i, 0)),
        out_shape=jax.ShapeDtypeStruct((np_, CH), F32),
    )


def _layer(xp, dstp, srcp, wq, bq, wk, bk, wv, bv, ws, bs,
           np_, s_per, nb, n_valid, act):
    q, k, v, s = _qkv_call(np_, xp.shape[1], n_valid, 512)(
        xp, wq, bq.reshape(1, -1), wk, bk.reshape(1, -1),
        wv, bv.reshape(1, -1), ws, bs.reshape(1, -1))
    den, msg = _sc_call(np_, s_per, nb)(
        q.reshape(H * np_, CH), k.reshape(H * np_, CH),
        v.reshape(H * NQ * np_, QW), dstp, srcp)
    return _fin_call(np_, 512, act)(
        msg.reshape(H, NQ, np_, QW), den.reshape(H, np_, QW), s)


def kernel(x, edge_index_list, Wq1, bq1, Wk1, bk1, Wv1, bv1, Ws1, bs1,
           Wq2, bq2, Wk2, bk2, Wv2, bv2, Ws2, bs2):
    B, N, Cin = x.shape
    E = B * edge_index_list.shape[2]
    offs = (jnp.arange(B, dtype=edge_index_list.dtype) * N)[:, None, None]
    flat = jnp.transpose(edge_index_list + offs, (1, 0, 2)).reshape(2, -1)
    src = flat[0].astype(I32)
    dst = flat[1].astype(I32)

    np_ = ((B * N + 2048) // 2048) * 2048
    s_per = ((E + NSUB * W - 1) // (NSUB * W)) * W
    nb = s_per // W
    ep = NSUB * s_per
    pad = jnp.full((ep - E,), B * N, I32)
    srcp = jnp.concatenate([src, pad])
    dstp = jnp.concatenate([dst, pad])
    xp = jnp.pad(x.reshape(B * N, Cin), ((0, np_ - B * N), (0, 0)))

    h = _layer(xp, dstp, srcp, Wq1, bq1, Wk1, bk1, Wv1, bv1, Ws1, bs1,
               np_, s_per, nb, B * N, act=True)
    o = _layer(h, dstp, srcp, Wq2, bq2, Wk2, bk2, Wv2, bv2, Ws2, bs2,
               np_, s_per, nb, B * N, act=False)
    return o[:B * N].reshape(B, N, CH)


# speedup vs baseline: 9.8755x; 1.0843x over previous
import dataclasses
import functools

import jax
import jax.numpy as jnp
from jax import lax
from jax.experimental import pallas as pl
from jax.experimental.pallas import tpu as pltpu
from jax.experimental.pallas import tpu_sc as plsc

F32 = jnp.float32
I32 = jnp.int32
LN = 16
H = 2
CH = 256
QW = 64
NQ = CH // QW
W = 64
NSUB = 16


def _qkv_body(x_ref, wq_ref, bq_ref, wk_ref, bk_ref, wv_ref, bv_ref,
              ws_ref, bs_ref, q_ref, k_ref, v_ref, s_ref, *, bn, n_valid):
    i = pl.program_id(0)
    rows = i * bn + lax.broadcasted_iota(I32, (bn, 1), 0)
    m = (rows < n_valid).astype(F32)
    x = x_ref[...] * m
    q = (jnp.dot(x, wq_ref[...], preferred_element_type=F32) + bq_ref[...]) * m
    k = (jnp.dot(x, wk_ref[...], preferred_element_type=F32) + bk_ref[...]) * m
    v = (jnp.dot(x, wv_ref[...], preferred_element_type=F32) + bv_ref[...]) * m
    s = (jnp.dot(x, ws_ref[...], preferred_element_type=F32) + bs_ref[...]) * m
    q_ref[0], q_ref[1] = q[:, :CH], q[:, CH:]
    k_ref[0], k_ref[1] = k[:, :CH], k[:, CH:]
    for qq in range(H * NQ):
        v_ref[qq] = v[:, qq * QW:(qq + 1) * QW]
    s_ref[...] = s


@functools.lru_cache(maxsize=None)
def _qkv_call(np_, cin, n_valid, bn):
    return pl.pallas_call(
        functools.partial(_qkv_body, bn=bn, n_valid=n_valid),
        grid=(np_ // bn,),
        in_specs=[
            pl.BlockSpec((bn, cin), lambda i: (i, 0)),
            pl.BlockSpec((cin, H * CH), lambda i: (0, 0)),
            pl.BlockSpec((1, H * CH), lambda i: (0, 0)),
            pl.BlockSpec((cin, H * CH), lambda i: (0, 0)),
            pl.BlockSpec((1, H * CH), lambda i: (0, 0)),
            pl.BlockSpec((cin, H * CH), lambda i: (0, 0)),
            pl.BlockSpec((1, H * CH), lambda i: (0, 0)),
            pl.BlockSpec((cin, CH), lambda i: (0, 0)),
            pl.BlockSpec((1, CH), lambda i: (0, 0)),
        ],
        out_specs=[
            pl.BlockSpec((H, bn, CH), lambda i: (0, i, 0)),
            pl.BlockSpec((H, bn, CH), lambda i: (0, i, 0)),
            pl.BlockSpec((H * NQ, bn, QW), lambda i: (0, i, 0)),
            pl.BlockSpec((bn, CH), lambda i: (i, 0)),
        ],
        out_shape=[
            jax.ShapeDtypeStruct((H, np_, CH), F32),
            jax.ShapeDtypeStruct((H, np_, CH), F32),
            jax.ShapeDtypeStruct((H * NQ, np_, QW), F32),
            jax.ShapeDtypeStruct((np_, CH), F32),
        ],
    )


def _sc_body(qtab, ktab, vtab, dst_hbm, src_hbm, den_out, msg_out, ex_out,
             dstb, srcb, idxq, idxs, qbuf, kbuf, vbuf, exb, albuf,
             zbuf, sq0, sq1, sk0, sk1, acc, *, np_, s_per, nb):
    c = lax.axis_index("c")
    t = lax.axis_index("s")
    ep = NSUB * s_per
    rpt = np_ // NSUB
    cnp = c * np_
    cep = c * ep
    lane = lax.iota(I32, LN)
    m_last = lane == (LN - 1)
    zeros = jnp.zeros((LN,), F32)
    zrows = zbuf.shape[0]
    semq = (sq0, sq1)
    semk = (sk0, sk1)

    @pl.loop(0, zrows)
    def _(r):
        for j in range(QW // LN):
            zbuf[r, pl.ds(j * LN, LN)] = zeros

    @pl.loop(0, rpt, step=zrows)
    def _(r):
        pltpu.sync_copy(zbuf, acc.at[pl.ds(t * rpt + r, zrows)])

    plsc.subcore_barrier()

    ebase0 = t * s_per

    def s1_issue(b, si):
        base = ebase0 + b * W
        pltpu.sync_copy(dst_hbm.at[pl.ds(base, W)], dstb.at[si])
        pltpu.sync_copy(src_hbm.at[pl.ds(base, W)], srcb.at[si])
        for j in range(W // LN):
            sl = pl.ds(j * LN, LN)
            idxq[si, sl] = dstb[si, sl] + cnp
            idxs[si, sl] = srcb[si, sl] + cnp
        pltpu.make_async_copy(qtab.at[idxq.at[si]], qbuf.at[si], semq[si]).start()
        pltpu.make_async_copy(ktab.at[idxs.at[si]], kbuf.at[si], semk[si]).start()

    def s1_compute(b, si):
        base = ebase0 + b * W
        pltpu.make_async_copy(qtab.at[idxq.at[si]], qbuf.at[si], semq[si]).wait()
        pltpu.make_async_copy(ktab.at[idxs.at[si]], kbuf.at[si], semk[si]).wait()

        @pl.loop(0, W)
        def _(e):
            a = qbuf[si, e, pl.ds(0, LN)] * kbuf[si, e, pl.ds(0, LN)]
            for j in range(1, CH // LN):
                sl = pl.ds(j * LN, LN)
                a = a + qbuf[si, e, sl] * kbuf[si, e, sl]
            plsc.store_scatter(albuf, [jnp.full((LN,), e, I32)],
                               plsc.cumsum(a), mask=m_last)

        for j in range(W // LN):
            sl = pl.ds(j * LN, LN)
            exb[si, sl] = jnp.exp(albuf[sl] * (1.0 / 16.0))
        pltpu.sync_copy(exb.at[si], ex_out.at[pl.ds(cep + base, W)])

        @pl.loop(0, W)
        def _(e):
            wv = plsc.load_gather(exb.at[si], [jnp.full((LN,), e, I32)])
            for j in range(QW // LN):
                vbuf[si, e, pl.ds(j * LN, LN)] = wv

        pltpu.sync_copy(vbuf.at[si], acc.at[dstb.at[si]], add=True)

    s1_issue(0, 0)

    @pl.loop(0, nb, step=2)
    def _(b):
        s1_issue(b + 1, 1)
        s1_compute(b, 0)

        @pl.when(b + 2 < nb)
        def _():
            s1_issue(b + 2, 0)

        s1_compute(b + 1, 1)

    plsc.subcore_barrier()
    pltpu.sync_copy(acc.at[pl.ds(t * rpt, rpt)],
                    den_out.at[pl.ds(cnp + t * rpt, rpt)])

    for quarter in range(NQ):
        voff = (NQ * c + quarter) * np_

        @pl.loop(0, rpt, step=zrows)
        def _(r):
            pltpu.sync_copy(zbuf, acc.at[pl.ds(t * rpt + r, zrows)])

        plsc.subcore_barrier()

        def sw_issue(b, si):
            base = ebase0 + b * W
            pltpu.sync_copy(dst_hbm.at[pl.ds(base, W)], dstb.at[si])
            pltpu.sync_copy(src_hbm.at[pl.ds(base, W)], srcb.at[si])
            pltpu.sync_copy(ex_out.at[pl.ds(cep + base, W)], exb.at[si])
            for j in range(W // LN):
                sl = pl.ds(j * LN, LN)
                idxs[si, sl] = srcb[si, sl] + voff
            pltpu.make_async_copy(vtab.at[idxs.at[si]], vbuf.at[si],
                                  semq[si]).start()

        def sw_compute(b, si):
            pltpu.make_async_copy(vtab.at[idxs.at[si]], vbuf.at[si],
                                  semq[si]).wait()

            @pl.loop(0, W)
            def _(e):
                wv = plsc.load_gather(exb.at[si], [jnp.full((LN,), e, I32)])
                for j in range(QW // LN):
                    sl = pl.ds(j * LN, LN)
                    vbuf[si, e, sl] = vbuf[si, e, sl] * wv

            pltpu.sync_copy(vbuf.at[si], acc.at[dstb.at[si]], add=True)

        sw_issue(0, 0)

        @pl.loop(0, nb, step=2)
        def _(b):
            sw_issue(b + 1, 1)
            sw_compute(b, 0)

            @pl.when(b + 2 < nb)
            def _():
                sw_issue(b + 2, 0)

            sw_compute(b + 1, 1)

        plsc.subcore_barrier()
        pltpu.sync_copy(acc.at[pl.ds(t * rpt, rpt)],
                        msg_out.at[pl.ds(voff + t * rpt, rpt)])
        plsc.subcore_barrier()


@functools.lru_cache(maxsize=None)
def _sc_call(np_, s_per, nb):
    mesh = plsc.VectorSubcoreMesh(core_axis_name="c", subcore_axis_name="s")
    cp = pltpu.CompilerParams()
    for fld, val in (("needs_layout_passes", False),
                     ("use_tc_tiling_on_sc", False)):
        if fld in pltpu.CompilerParams.__dataclass_fields__:
            cp = dataclasses.replace(cp, **{fld: val})
    return pl.kernel(
        functools.partial(_sc_body, np_=np_, s_per=s_per, nb=nb),
        out_type=(
            jax.ShapeDtypeStruct((H * np_, QW), F32),
            jax.ShapeDtypeStruct((H * NQ * np_, QW), F32),
            jax.ShapeDtypeStruct((H * NSUB * s_per,), F32),
        ),
        mesh=mesh,
        compiler_params=cp,
        scratch_types=[
            pltpu.VMEM((2, W), I32),
            pltpu.VMEM((2, W), I32),
            pltpu.VMEM((2, W), I32),
            pltpu.VMEM((2, W), I32),
            pltpu.VMEM((2, W, CH), F32),
            pltpu.VMEM((2, W, CH), F32),
            pltpu.VMEM((2, W, QW), F32),
            pltpu.VMEM((2, W), F32),
            pltpu.VMEM((W,), F32),
            pltpu.VMEM((32, QW), F32),
            pltpu.SemaphoreType.DMA,
            pltpu.SemaphoreType.DMA,
            pltpu.SemaphoreType.DMA,
            pltpu.SemaphoreType.DMA,
            pltpu.VMEM_SHARED((np_, QW), F32),
        ],
    )


def _fin_body(msg_ref, den_ref, s_ref, o_ref, *, act):
    o = s_ref[...]
    ms = []
    for h in range(H):
        d = den_ref[h, :, 0:1] + 1e-16
        m = jnp.concatenate([msg_ref[h, qq] for qq in range(NQ)], axis=1) / d
        ms.append(m)
    o = o + sum(ms) * (1.0 / H)
    if act:
        o = jnp.where(o >= 0, o, 0.1 * o)
    o_ref[...] = o


@functools.lru_cache(maxsize=None)
def _fin_call(np_, bn, act):
    return pl.pallas_call(
        functools.partial(_fin_body, act=act),
        grid=(np_ // bn,),
        in_specs=[
            pl.BlockSpec((H, NQ, bn, QW), lambda i: (0, 0, i, 0)),
            pl.BlockSpec((H, bn, QW), lambda i: (0, i, 0)),
            pl.BlockSpec((bn, CH), lambda i: (i, 0)),
        ],
        out_specs=pl.BlockSpec((bn, CH), lambda i: (i, 0)),
        out_shape=jax.ShapeDtypeStruct((np_, CH), F32),
    )


def _layer(xp, dstp, srcp, wq, bq, wk, bk, wv, bv, ws, bs,
           np_, s_per, nb, n_valid, act):
    q, k, v, s = _qkv_call(np_, xp.shape[1], n_valid, 512)(
        xp, wq, bq.reshape(1, -1), wk, bk.reshape(1, -1),
        wv, bv.reshape(1, -1), ws, bs.reshape(1, -1))
    den, msg, _ = _sc_call(np_, s_per, nb)(
        q.reshape(H * np_, CH), k.reshape(H * np_, CH),
        v.reshape(H * NQ * np_, QW), dstp, srcp)
    return _fin_call(np_, 512, act)(
        msg.reshape(H, NQ, np_, QW), den.reshape(H, np_, QW), s)


def kernel(x, edge_index_list, Wq1, bq1, Wk1, bk1, Wv1, bv1, Ws1, bs1,
           Wq2, bq2, Wk2, bk2, Wv2, bv2, Ws2, bs2):
    B, N, Cin = x.shape
    E = B * edge_index_list.shape[2]
    offs = (jnp.arange(B, dtype=edge_index_list.dtype) * N)[:, None, None]
    flat = jnp.transpose(edge_index_list + offs, (1, 0, 2)).reshape(2, -1)
    src = flat[0].astype(I32)
    dst = flat[1].astype(I32)

    np_ = ((B * N + 2048) // 2048) * 2048
    s_per = ((E + NSUB * 2 * W - 1) // (NSUB * 2 * W)) * 2 * W
    nb = s_per // W
    ep = NSUB * s_per
    pad = jnp.full((ep - E,), B * N, I32)
    srcp = jnp.concatenate([src, pad])
    dstp = jnp.concatenate([dst, pad])
    xp = jnp.pad(x.reshape(B * N, Cin), ((0, np_ - B * N), (0, 0)))

    h = _layer(xp, dstp, srcp, Wq1, bq1, Wk1, bk1, Wv1, bv1, Ws1, bs1,
               np_, s_per, nb, B * N, act=True)
    o = _layer(h, dstp, srcp, Wq2, bq2, Wk2, bk2, Wv2, bv2, Ws2, bs2,
               np_, s_per, nb, B * N, act=False)
    return o[:B * N].reshape(B, N, CH)


# speedup vs baseline: 11.6086x; 1.1755x over previous
import dataclasses
import functools

import jax
import jax.numpy as jnp
from jax import lax
from jax.experimental import pallas as pl
from jax.experimental.pallas import tpu as pltpu
from jax.experimental.pallas import tpu_sc as plsc

F32 = jnp.float32
I32 = jnp.int32
LN = 16
H = 2
CH = 256
QW = 64
NQ = CH // QW
W = 64
NSUB = 16


def _qkv_body(x_ref, wq_ref, bq_ref, wk_ref, bk_ref, wv_ref, bv_ref,
              ws_ref, bs_ref, q_ref, k_ref, v_ref, s_ref, *, bn, n_valid):
    i = pl.program_id(0)
    rows = i * bn + lax.broadcasted_iota(I32, (bn, 1), 0)
    m = (rows < n_valid).astype(F32)
    x = x_ref[...] * m
    q = (jnp.dot(x, wq_ref[...], preferred_element_type=F32) + bq_ref[...]) * m
    k = (jnp.dot(x, wk_ref[...], preferred_element_type=F32) + bk_ref[...]) * m
    v = (jnp.dot(x, wv_ref[...], preferred_element_type=F32) + bv_ref[...]) * m
    s = (jnp.dot(x, ws_ref[...], preferred_element_type=F32) + bs_ref[...]) * m
    q_ref[0], q_ref[1] = q[:, :CH], q[:, CH:]
    k_ref[0], k_ref[1] = k[:, :CH], k[:, CH:]
    for qq in range(H * NQ):
        v_ref[qq] = v[:, qq * QW:(qq + 1) * QW]
    s_ref[...] = s


@functools.lru_cache(maxsize=None)
def _qkv_call(np_, cin, n_valid, bn):
    return pl.pallas_call(
        functools.partial(_qkv_body, bn=bn, n_valid=n_valid),
        grid=(np_ // bn,),
        in_specs=[
            pl.BlockSpec((bn, cin), lambda i: (i, 0)),
            pl.BlockSpec((cin, H * CH), lambda i: (0, 0)),
            pl.BlockSpec((1, H * CH), lambda i: (0, 0)),
            pl.BlockSpec((cin, H * CH), lambda i: (0, 0)),
            pl.BlockSpec((1, H * CH), lambda i: (0, 0)),
            pl.BlockSpec((cin, H * CH), lambda i: (0, 0)),
            pl.BlockSpec((1, H * CH), lambda i: (0, 0)),
            pl.BlockSpec((cin, CH), lambda i: (0, 0)),
            pl.BlockSpec((1, CH), lambda i: (0, 0)),
        ],
        out_specs=[
            pl.BlockSpec((H, bn, CH), lambda i: (0, i, 0)),
            pl.BlockSpec((H, bn, CH), lambda i: (0, i, 0)),
            pl.BlockSpec((H * NQ, bn, QW), lambda i: (0, i, 0)),
            pl.BlockSpec((bn, CH), lambda i: (i, 0)),
        ],
        out_shape=[
            jax.ShapeDtypeStruct((H, np_, CH), F32),
            jax.ShapeDtypeStruct((H, np_, CH), F32),
            jax.ShapeDtypeStruct((H * NQ, np_, QW), F32),
            jax.ShapeDtypeStruct((np_, CH), F32),
        ],
    )


def _sc_body(qtab, ktab, vtab, dst_hbm, src_hbm, den_out, msg_out, ex_out,
             dstb, srcb, idxq, idxs, qbuf, kbuf, vbuf, exb, albuf,
             zbuf, sq0, sq1, sk0, sk1, acc, *, np_, s_per, nb):
    c = lax.axis_index("c")
    t = lax.axis_index("s")
    ep = NSUB * s_per
    rpt = np_ // NSUB
    cnp = c * np_
    cep = c * ep
    lane = lax.iota(I32, LN)
    m_last = lane == (LN - 1)
    zeros = jnp.zeros((LN,), F32)
    zrows = zbuf.shape[0]
    semq = (sq0, sq1)
    semk = (sk0, sk1)

    @pl.loop(0, zrows)
    def _(r):
        for j in range(QW // LN):
            zbuf[r, pl.ds(j * LN, LN)] = zeros

    @pl.loop(0, rpt, step=zrows)
    def _(r):
        pltpu.sync_copy(zbuf, acc.at[pl.ds(t * rpt + r, zrows)])

    plsc.subcore_barrier()

    ebase0 = t * s_per

    def s1_issue(b, si):
        base = ebase0 + b * W
        pltpu.sync_copy(dst_hbm.at[pl.ds(base, W)], dstb.at[si])
        pltpu.sync_copy(src_hbm.at[pl.ds(base, W)], srcb.at[si])
        for j in range(W // LN):
            sl = pl.ds(j * LN, LN)
            idxq[si, sl] = dstb[si, sl] + cnp
            idxs[si, sl] = srcb[si, sl] + cnp
        pltpu.make_async_copy(qtab.at[idxq.at[si]], qbuf.at[si], semq[si]).start()
        pltpu.make_async_copy(ktab.at[idxs.at[si]], kbuf.at[si], semk[si]).start()

    def s1_compute(b, si):
        base = ebase0 + b * W
        pltpu.make_async_copy(qtab.at[idxq.at[si]], qbuf.at[si], semq[si]).wait()
        pltpu.make_async_copy(ktab.at[idxs.at[si]], kbuf.at[si], semk[si]).wait()

        @plsc.parallel_loop(0, W, unroll=4)
        def _(e):
            a = qbuf[si, e, pl.ds(0, LN)] * kbuf[si, e, pl.ds(0, LN)]
            for j in range(1, CH // LN):
                sl = pl.ds(j * LN, LN)
                a = a + qbuf[si, e, sl] * kbuf[si, e, sl]
            plsc.store_scatter(albuf, [jnp.full((LN,), e, I32)],
                               plsc.cumsum(a), mask=m_last)

        for j in range(W // LN):
            sl = pl.ds(j * LN, LN)
            exb[si, sl] = jnp.exp(albuf[sl] * (1.0 / 16.0))
        pltpu.sync_copy(exb.at[si], ex_out.at[pl.ds(cep + base, W)])

        @plsc.parallel_loop(0, W, unroll=4)
        def _(e):
            wv = plsc.load_gather(exb.at[si], [jnp.full((LN,), e, I32)])
            for j in range(QW // LN):
                vbuf[si, e, pl.ds(j * LN, LN)] = wv

        pltpu.sync_copy(vbuf.at[si], acc.at[dstb.at[si]], add=True)

    s1_issue(0, 0)

    @pl.loop(0, nb, step=2)
    def _(b):
        s1_issue(b + 1, 1)
        s1_compute(b, 0)

        @pl.when(b + 2 < nb)
        def _():
            s1_issue(b + 2, 0)

        s1_compute(b + 1, 1)

    plsc.subcore_barrier()
    pltpu.sync_copy(acc.at[pl.ds(t * rpt, rpt)],
                    den_out.at[pl.ds(cnp + t * rpt, rpt)])

    for quarter in range(NQ):
        voff = (NQ * c + quarter) * np_

        @pl.loop(0, rpt, step=zrows)
        def _(r):
            pltpu.sync_copy(zbuf, acc.at[pl.ds(t * rpt + r, zrows)])

        plsc.subcore_barrier()

        def sw_issue(b, si):
            base = ebase0 + b * W
            pltpu.sync_copy(dst_hbm.at[pl.ds(base, W)], dstb.at[si])
            pltpu.sync_copy(src_hbm.at[pl.ds(base, W)], srcb.at[si])
            pltpu.sync_copy(ex_out.at[pl.ds(cep + base, W)], exb.at[si])
            for j in range(W // LN):
                sl = pl.ds(j * LN, LN)
                idxs[si, sl] = srcb[si, sl] + voff
            pltpu.make_async_copy(vtab.at[idxs.at[si]], vbuf.at[si],
                                  semq[si]).start()

        def sw_compute(b, si):
            pltpu.make_async_copy(vtab.at[idxs.at[si]], vbuf.at[si],
                                  semq[si]).wait()

            @plsc.parallel_loop(0, W, unroll=4)
            def _(e):
                wv = plsc.load_gather(exb.at[si], [jnp.full((LN,), e, I32)])
                for j in range(QW // LN):
                    sl = pl.ds(j * LN, LN)
                    vbuf[si, e, sl] = vbuf[si, e, sl] * wv

            pltpu.sync_copy(vbuf.at[si], acc.at[dstb.at[si]], add=True)

        sw_issue(0, 0)

        @pl.loop(0, nb, step=2)
        def _(b):
            sw_issue(b + 1, 1)
            sw_compute(b, 0)

            @pl.when(b + 2 < nb)
            def _():
                sw_issue(b + 2, 0)

            sw_compute(b + 1, 1)

        plsc.subcore_barrier()
        pltpu.sync_copy(acc.at[pl.ds(t * rpt, rpt)],
                        msg_out.at[pl.ds(voff + t * rpt, rpt)])
        plsc.subcore_barrier()


@functools.lru_cache(maxsize=None)
def _sc_call(np_, s_per, nb):
    mesh = plsc.VectorSubcoreMesh(core_axis_name="c", subcore_axis_name="s")
    cp = pltpu.CompilerParams()
    for fld, val in (("needs_layout_passes", False),
                     ("use_tc_tiling_on_sc", False)):
        if fld in pltpu.CompilerParams.__dataclass_fields__:
            cp = dataclasses.replace(cp, **{fld: val})
    return pl.kernel(
        functools.partial(_sc_body, np_=np_, s_per=s_per, nb=nb),
        out_type=(
            jax.ShapeDtypeStruct((H * np_, QW), F32),
            jax.ShapeDtypeStruct((H * NQ * np_, QW), F32),
            jax.ShapeDtypeStruct((H * NSUB * s_per,), F32),
        ),
        mesh=mesh,
        compiler_params=cp,
        scratch_types=[
            pltpu.VMEM((2, W), I32),
            pltpu.VMEM((2, W), I32),
            pltpu.VMEM((2, W), I32),
            pltpu.VMEM((2, W), I32),
            pltpu.VMEM((2, W, CH), F32),
            pltpu.VMEM((2, W, CH), F32),
            pltpu.VMEM((2, W, QW), F32),
            pltpu.VMEM((2, W), F32),
            pltpu.VMEM((W,), F32),
            pltpu.VMEM((32, QW), F32),
            pltpu.SemaphoreType.DMA,
            pltpu.SemaphoreType.DMA,
            pltpu.SemaphoreType.DMA,
            pltpu.SemaphoreType.DMA,
            pltpu.VMEM_SHARED((np_, QW), F32),
        ],
    )


def _fin_body(msg_ref, den_ref, s_ref, o_ref, *, act):
    o = s_ref[...]
    ms = []
    for h in range(H):
        d = den_ref[h, :, 0:1] + 1e-16
        m = jnp.concatenate([msg_ref[h, qq] for qq in range(NQ)], axis=1) / d
        ms.append(m)
    o = o + sum(ms) * (1.0 / H)
    if act:
        o = jnp.where(o >= 0, o, 0.1 * o)
    o_ref[...] = o


@functools.lru_cache(maxsize=None)
def _fin_call(np_, bn, act):
    return pl.pallas_call(
        functools.partial(_fin_body, act=act),
        grid=(np_ // bn,),
        in_specs=[
            pl.BlockSpec((H, NQ, bn, QW), lambda i: (0, 0, i, 0)),
            pl.BlockSpec((H, bn, QW), lambda i: (0, i, 0)),
            pl.BlockSpec((bn, CH), lambda i: (i, 0)),
        ],
        out_specs=pl.BlockSpec((bn, CH), lambda i: (i, 0)),
        out_shape=jax.ShapeDtypeStruct((np_, CH), F32),
    )


def _layer(xp, dstp, srcp, wq, bq, wk, bk, wv, bv, ws, bs,
           np_, s_per, nb, n_valid, act):
    q, k, v, s = _qkv_call(np_, xp.shape[1], n_valid, 512)(
        xp, wq, bq.reshape(1, -1), wk, bk.reshape(1, -1),
        wv, bv.reshape(1, -1), ws, bs.reshape(1, -1))
    den, msg, _ = _sc_call(np_, s_per, nb)(
        q.reshape(H * np_, CH), k.reshape(H * np_, CH),
        v.reshape(H * NQ * np_, QW), dstp, srcp)
    return _fin_call(np_, 512, act)(
        msg.reshape(H, NQ, np_, QW), den.reshape(H, np_, QW), s)


def kernel(x, edge_index_list, Wq1, bq1, Wk1, bk1, Wv1, bv1, Ws1, bs1,
           Wq2, bq2, Wk2, bk2, Wv2, bv2, Ws2, bs2):
    B, N, Cin = x.shape
    E = B * edge_index_list.shape[2]
    offs = (jnp.arange(B, dtype=edge_index_list.dtype) * N)[:, None, None]
    flat = jnp.transpose(edge_index_list + offs, (1, 0, 2)).reshape(2, -1)
    src = flat[0].astype(I32)
    dst = flat[1].astype(I32)

    np_ = ((B * N + 2048) // 2048) * 2048
    s_per = ((E + NSUB * 2 * W - 1) // (NSUB * 2 * W)) * 2 * W
    nb = s_per // W
    ep = NSUB * s_per
    pad = jnp.full((ep - E,), B * N, I32)
    srcp = jnp.concatenate([src, pad])
    dstp = jnp.concatenate([dst, pad])
    xp = jnp.pad(x.reshape(B * N, Cin), ((0, np_ - B * N), (0, 0)))

    h = _layer(xp, dstp, srcp, Wq1, bq1, Wk1, bk1, Wv1, bv1, Ws1, bs1,
               np_, s_per, nb, B * N, act=True)
    o = _layer(h, dstp, srcp, Wq2, bq2, Wk2, bk2, Wv2, bv2, Ws2, bs2,
               np_, s_per, nb, B * N, act=False)
    return o[:B * N].reshape(B, N, CH)


# speedup vs baseline: 12.1771x; 1.0490x over previous
import dataclasses
import functools

import jax
import jax.numpy as jnp
from jax import lax
from jax.experimental import pallas as pl
from jax.experimental.pallas import tpu as pltpu
from jax.experimental.pallas import tpu_sc as plsc

F32 = jnp.float32
BF16 = jnp.bfloat16
I32 = jnp.int32
LN = 16
H = 2
CH = 256
QW = 64
NQ = CH // QW
W = 64
NSUB = 16


def _qkv_body(x_ref, wq_ref, bq_ref, wk_ref, bk_ref, wv_ref, bv_ref,
              ws_ref, bs_ref, q_ref, k_ref, v_ref, s_ref, *, bn, n_valid):
    i = pl.program_id(0)
    rows = i * bn + lax.broadcasted_iota(I32, (bn, 1), 0)
    m = (rows < n_valid).astype(F32)
    x = x_ref[...] * m
    q = (jnp.dot(x, wq_ref[...], preferred_element_type=F32) + bq_ref[...]) * m
    k = (jnp.dot(x, wk_ref[...], preferred_element_type=F32) + bk_ref[...]) * m
    v = (jnp.dot(x, wv_ref[...], preferred_element_type=F32) + bv_ref[...]) * m
    s = (jnp.dot(x, ws_ref[...], preferred_element_type=F32) + bs_ref[...]) * m
    q16, k16 = q.astype(BF16), k.astype(BF16)
    q_ref[0], q_ref[1] = q16[:, :CH], q16[:, CH:]
    k_ref[0], k_ref[1] = k16[:, :CH], k16[:, CH:]
    for qq in range(H * NQ):
        v_ref[qq] = v[:, qq * QW:(qq + 1) * QW]
    s_ref[...] = s


@functools.lru_cache(maxsize=None)
def _qkv_call(np_, cin, n_valid, bn):
    return pl.pallas_call(
        functools.partial(_qkv_body, bn=bn, n_valid=n_valid),
        grid=(np_ // bn,),
        in_specs=[
            pl.BlockSpec((bn, cin), lambda i: (i, 0)),
            pl.BlockSpec((cin, H * CH), lambda i: (0, 0)),
            pl.BlockSpec((1, H * CH), lambda i: (0, 0)),
            pl.BlockSpec((cin, H * CH), lambda i: (0, 0)),
            pl.BlockSpec((1, H * CH), lambda i: (0, 0)),
            pl.BlockSpec((cin, H * CH), lambda i: (0, 0)),
            pl.BlockSpec((1, H * CH), lambda i: (0, 0)),
            pl.BlockSpec((cin, CH), lambda i: (0, 0)),
            pl.BlockSpec((1, CH), lambda i: (0, 0)),
        ],
        out_specs=[
            pl.BlockSpec((H, bn, CH), lambda i: (0, i, 0)),
            pl.BlockSpec((H, bn, CH), lambda i: (0, i, 0)),
            pl.BlockSpec((H * NQ, bn, QW), lambda i: (0, i, 0)),
            pl.BlockSpec((bn, CH), lambda i: (i, 0)),
        ],
        out_shape=[
            jax.ShapeDtypeStruct((H, np_, CH), BF16),
            jax.ShapeDtypeStruct((H, np_, CH), BF16),
            jax.ShapeDtypeStruct((H * NQ, np_, QW), F32),
            jax.ShapeDtypeStruct((np_, CH), F32),
        ],
    )


def _sc_body(qtab, ktab, vtab, dst_hbm, src_hbm, den_out, msg_out, ex_out,
             dstb, srcb, idxq, idxs, qbuf, kbuf, vbuf, exb, albuf,
             zbuf, sq0, sq1, sk0, sk1, acc, *, np_, s_per, nb):
    c = lax.axis_index("c")
    t = lax.axis_index("s")
    ep = NSUB * s_per
    rpt = np_ // NSUB
    cnp = c * np_
    cep = c * ep
    lane = lax.iota(I32, LN)
    m_last = lane == (LN - 1)
    zeros = jnp.zeros((LN,), F32)
    zrows = zbuf.shape[0]
    semq = (sq0, sq1)
    semk = (sk0, sk1)

    @pl.loop(0, zrows)
    def _(r):
        for j in range(QW // LN):
            zbuf[r, pl.ds(j * LN, LN)] = zeros

    @pl.loop(0, rpt, step=zrows)
    def _(r):
        pltpu.sync_copy(zbuf, acc.at[pl.ds(t * rpt + r, zrows)])

    plsc.subcore_barrier()

    ebase0 = t * s_per

    def s1_issue(b, si):
        base = ebase0 + b * W
        pltpu.sync_copy(dst_hbm.at[pl.ds(base, W)], dstb.at[si])
        pltpu.sync_copy(src_hbm.at[pl.ds(base, W)], srcb.at[si])
        for j in range(W // LN):
            sl = pl.ds(j * LN, LN)
            idxq[si, sl] = dstb[si, sl] + cnp
            idxs[si, sl] = srcb[si, sl] + cnp
        pltpu.make_async_copy(qtab.at[idxq.at[si]], qbuf.at[si], semq[si]).start()
        pltpu.make_async_copy(ktab.at[idxs.at[si]], kbuf.at[si], semk[si]).start()

    def s1_compute(b, si):
        base = ebase0 + b * W
        pltpu.make_async_copy(qtab.at[idxq.at[si]], qbuf.at[si], semq[si]).wait()
        pltpu.make_async_copy(ktab.at[idxs.at[si]], kbuf.at[si], semk[si]).wait()

        @plsc.parallel_loop(0, W, unroll=4)
        def _(e):
            a = jnp.zeros((LN,), F32)
            for j in range(CH // (2 * LN)):
                sl = pl.ds(j * 2 * LN, 2 * LN)
                qa, qb = plsc.unpack(qbuf[si, e, sl],
                                     format=plsc.PackFormat.INTERLEAVED,
                                     preferred_element_type=F32)
                ka, kb = plsc.unpack(kbuf[si, e, sl],
                                     format=plsc.PackFormat.INTERLEAVED,
                                     preferred_element_type=F32)
                a = a + qa * ka + qb * kb
            plsc.store_scatter(albuf, [jnp.full((LN,), e, I32)],
                               plsc.cumsum(a), mask=m_last)

        for j in range(W // LN):
            sl = pl.ds(j * LN, LN)
            exb[si, sl] = jnp.exp(albuf[sl] * (1.0 / 16.0))
        pltpu.sync_copy(exb.at[si], ex_out.at[pl.ds(cep + base, W)])

        @plsc.parallel_loop(0, W, unroll=4)
        def _(e):
            wv = plsc.load_gather(exb.at[si], [jnp.full((LN,), e, I32)])
            for j in range(QW // LN):
                vbuf[si, e, pl.ds(j * LN, LN)] = wv

        pltpu.sync_copy(vbuf.at[si], acc.at[dstb.at[si]], add=True)

    s1_issue(0, 0)

    @pl.loop(0, nb, step=2)
    def _(b):
        s1_issue(b + 1, 1)
        s1_compute(b, 0)

        @pl.when(b + 2 < nb)
        def _():
            s1_issue(b + 2, 0)

        s1_compute(b + 1, 1)

    plsc.subcore_barrier()
    pltpu.sync_copy(acc.at[pl.ds(t * rpt, rpt)],
                    den_out.at[pl.ds(cnp + t * rpt, rpt)])

    for quarter in range(NQ):
        voff = (NQ * c + quarter) * np_

        @pl.loop(0, rpt, step=zrows)
        def _(r):
            pltpu.sync_copy(zbuf, acc.at[pl.ds(t * rpt + r, zrows)])

        plsc.subcore_barrier()

        def sw_issue(b, si):
            base = ebase0 + b * W
            pltpu.sync_copy(dst_hbm.at[pl.ds(base, W)], dstb.at[si])
            pltpu.sync_copy(src_hbm.at[pl.ds(base, W)], srcb.at[si])
            pltpu.sync_copy(ex_out.at[pl.ds(cep + base, W)], exb.at[si])
            for j in range(W // LN):
                sl = pl.ds(j * LN, LN)
                idxs[si, sl] = srcb[si, sl] + voff
            pltpu.make_async_copy(vtab.at[idxs.at[si]], vbuf.at[si],
                                  semq[si]).start()

        def sw_compute(b, si):
            pltpu.make_async_copy(vtab.at[idxs.at[si]], vbuf.at[si],
                                  semq[si]).wait()

            @plsc.parallel_loop(0, W, unroll=4)
            def _(e):
                wv = plsc.load_gather(exb.at[si], [jnp.full((LN,), e, I32)])
                for j in range(QW // LN):
                    sl = pl.ds(j * LN, LN)
                    vbuf[si, e, sl] = vbuf[si, e, sl] * wv

            pltpu.sync_copy(vbuf.at[si], acc.at[dstb.at[si]], add=True)

        sw_issue(0, 0)

        @pl.loop(0, nb, step=2)
        def _(b):
            sw_issue(b + 1, 1)
            sw_compute(b, 0)

            @pl.when(b + 2 < nb)
            def _():
                sw_issue(b + 2, 0)

            sw_compute(b + 1, 1)

        plsc.subcore_barrier()
        pltpu.sync_copy(acc.at[pl.ds(t * rpt, rpt)],
                        msg_out.at[pl.ds(voff + t * rpt, rpt)])
        plsc.subcore_barrier()


@functools.lru_cache(maxsize=None)
def _sc_call(np_, s_per, nb):
    mesh = plsc.VectorSubcoreMesh(core_axis_name="c", subcore_axis_name="s")
    cp = pltpu.CompilerParams()
    for fld, val in (("needs_layout_passes", False),
                     ("use_tc_tiling_on_sc", False)):
        if fld in pltpu.CompilerParams.__dataclass_fields__:
            cp = dataclasses.replace(cp, **{fld: val})
    return pl.kernel(
        functools.partial(_sc_body, np_=np_, s_per=s_per, nb=nb),
        out_type=(
            jax.ShapeDtypeStruct((H * np_, QW), F32),
            jax.ShapeDtypeStruct((H * NQ * np_, QW), F32),
            jax.ShapeDtypeStruct((H * NSUB * s_per,), F32),
        ),
        mesh=mesh,
        compiler_params=cp,
        scratch_types=[
            pltpu.VMEM((2, W), I32),
            pltpu.VMEM((2, W), I32),
            pltpu.VMEM((2, W), I32),
            pltpu.VMEM((2, W), I32),
            pltpu.VMEM((2, W, CH), BF16),
            pltpu.VMEM((2, W, CH), BF16),
            pltpu.VMEM((2, W, QW), F32),
            pltpu.VMEM((2, W), F32),
            pltpu.VMEM((W,), F32),
            pltpu.VMEM((32, QW), F32),
            pltpu.SemaphoreType.DMA,
            pltpu.SemaphoreType.DMA,
            pltpu.SemaphoreType.DMA,
            pltpu.SemaphoreType.DMA,
            pltpu.VMEM_SHARED((np_, QW), F32),
        ],
    )


def _fin_body(msg_ref, den_ref, s_ref, o_ref, *, act):
    o = s_ref[...]
    ms = []
    for h in range(H):
        d = den_ref[h, :, 0:1] + 1e-16
        m = jnp.concatenate([msg_ref[h, qq] for qq in range(NQ)], axis=1) / d
        ms.append(m)
    o = o + sum(ms) * (1.0 / H)
    if act:
        o = jnp.where(o >= 0, o, 0.1 * o)
    o_ref[...] = o


@functools.lru_cache(maxsize=None)
def _fin_call(np_, bn, act):
    return pl.pallas_call(
        functools.partial(_fin_body, act=act),
        grid=(np_ // bn,),
        in_specs=[
            pl.BlockSpec((H, NQ, bn, QW), lambda i: (0, 0, i, 0)),
            pl.BlockSpec((H, bn, QW), lambda i: (0, i, 0)),
            pl.BlockSpec((bn, CH), lambda i: (i, 0)),
        ],
        out_specs=pl.BlockSpec((bn, CH), lambda i: (i, 0)),
        out_shape=jax.ShapeDtypeStruct((np_, CH), F32),
    )


def _layer(xp, dstp, srcp, wq, bq, wk, bk, wv, bv, ws, bs,
           np_, s_per, nb, n_valid, act):
    q, k, v, s = _qkv_call(np_, xp.shape[1], n_valid, 512)(
        xp, wq, bq.reshape(1, -1), wk, bk.reshape(1, -1),
        wv, bv.reshape(1, -1), ws, bs.reshape(1, -1))
    den, msg, _ = _sc_call(np_, s_per, nb)(
        q.reshape(H * np_, CH), k.reshape(H * np_, CH),
        v.reshape(H * NQ * np_, QW), dstp, srcp)
    return _fin_call(np_, 512, act)(
        msg.reshape(H, NQ, np_, QW), den.reshape(H, np_, QW), s)


def kernel(x, edge_index_list, Wq1, bq1, Wk1, bk1, Wv1, bv1, Ws1, bs1,
           Wq2, bq2, Wk2, bk2, Wv2, bv2, Ws2, bs2):
    B, N, Cin = x.shape
    E = B * edge_index_list.shape[2]
    offs = (jnp.arange(B, dtype=edge_index_list.dtype) * N)[:, None, None]
    flat = jnp.transpose(edge_index_list + offs, (1, 0, 2)).reshape(2, -1)
    src = flat[0].astype(I32)
    dst = flat[1].astype(I32)

    np_ = ((B * N + 2048) // 2048) * 2048
    s_per = ((E + NSUB * 2 * W - 1) // (NSUB * 2 * W)) * 2 * W
    nb = s_per // W
    ep = NSUB * s_per
    pad = jnp.full((ep - E,), B * N, I32)
    srcp = jnp.concatenate([src, pad])
    dstp = jnp.concatenate([dst, pad])
    xp = jnp.pad(x.reshape(B * N, Cin), ((0, np_ - B * N), (0, 0)))

    h = _layer(xp, dstp, srcp, Wq1, bq1, Wk1, bk1, Wv1, bv1, Ws1, bs1,
               np_, s_per, nb, B * N, act=True)
    o = _layer(h, dstp, srcp, Wq2, bq2, Wk2, bk2, Wv2, bv2, Ws2, bs2,
               np_, s_per, nb, B * N, act=False)
    return o[:B * N].reshape(B, N, CH)


# speedup vs baseline: 13.9511x; 1.1457x over previous
import dataclasses
import functools

import jax
import jax.numpy as jnp
from jax import lax
from jax.experimental import pallas as pl
from jax.experimental.pallas import tpu as pltpu
from jax.experimental.pallas import tpu_sc as plsc

F32 = jnp.float32
BF16 = jnp.bfloat16
I32 = jnp.int32
LN = 16
H = 2
CH = 256
QW = 64
NQ = CH // QW
W = 128
NSUB = 16


def _qkv_body(x_ref, wq_ref, bq_ref, wk_ref, bk_ref, wv_ref, bv_ref,
              ws_ref, bs_ref, q_ref, k_ref, v_ref, s_ref, *, bn, n_valid):
    i = pl.program_id(0)
    rows = i * bn + lax.broadcasted_iota(I32, (bn, 1), 0)
    m = (rows < n_valid).astype(F32)
    x = x_ref[...] * m
    q = (jnp.dot(x, wq_ref[...], preferred_element_type=F32) + bq_ref[...]) * m
    k = (jnp.dot(x, wk_ref[...], preferred_element_type=F32) + bk_ref[...]) * m
    v = (jnp.dot(x, wv_ref[...], preferred_element_type=F32) + bv_ref[...]) * m
    s = (jnp.dot(x, ws_ref[...], preferred_element_type=F32) + bs_ref[...]) * m
    q16, k16 = q.astype(BF16), k.astype(BF16)
    q_ref[0], q_ref[1] = q16[:, :CH], q16[:, CH:]
    k_ref[0], k_ref[1] = k16[:, :CH], k16[:, CH:]
    for qq in range(H * NQ):
        v_ref[qq] = v[:, qq * QW:(qq + 1) * QW]
    s_ref[...] = s


@functools.lru_cache(maxsize=None)
def _qkv_call(np_, cin, n_valid, bn):
    return pl.pallas_call(
        functools.partial(_qkv_body, bn=bn, n_valid=n_valid),
        grid=(np_ // bn,),
        in_specs=[
            pl.BlockSpec((bn, cin), lambda i: (i, 0)),
            pl.BlockSpec((cin, H * CH), lambda i: (0, 0)),
            pl.BlockSpec((1, H * CH), lambda i: (0, 0)),
            pl.BlockSpec((cin, H * CH), lambda i: (0, 0)),
            pl.BlockSpec((1, H * CH), lambda i: (0, 0)),
            pl.BlockSpec((cin, H * CH), lambda i: (0, 0)),
            pl.BlockSpec((1, H * CH), lambda i: (0, 0)),
            pl.BlockSpec((cin, CH), lambda i: (0, 0)),
            pl.BlockSpec((1, CH), lambda i: (0, 0)),
        ],
        out_specs=[
            pl.BlockSpec((H, bn, CH), lambda i: (0, i, 0)),
            pl.BlockSpec((H, bn, CH), lambda i: (0, i, 0)),
            pl.BlockSpec((H * NQ, bn, QW), lambda i: (0, i, 0)),
            pl.BlockSpec((bn, CH), lambda i: (i, 0)),
        ],
        out_shape=[
            jax.ShapeDtypeStruct((H, np_, CH), BF16),
            jax.ShapeDtypeStruct((H, np_, CH), BF16),
            jax.ShapeDtypeStruct((H * NQ, np_, QW), F32),
            jax.ShapeDtypeStruct((np_, CH), F32),
        ],
    )


def _sc_body(qtab, ktab, vtab, dst_hbm, src_hbm, den_out, msg_out, ex_out,
             dstb, srcb, idxq, idxs, qbuf, kbuf, vbuf, exb, albuf,
             zbuf, sq0, sq1, sk0, sk1, acc, *, np_, s_per, nb):
    c = lax.axis_index("c")
    t = lax.axis_index("s")
    ep = NSUB * s_per
    rpt = np_ // NSUB
    cnp = c * np_
    cep = c * ep
    lane = lax.iota(I32, LN)
    m_last = lane == (LN - 1)
    zeros = jnp.zeros((LN,), F32)
    zrows = zbuf.shape[0]
    semq = (sq0, sq1)
    semk = (sk0, sk1)

    @pl.loop(0, zrows)
    def _(r):
        for j in range(QW // LN):
            zbuf[r, pl.ds(j * LN, LN)] = zeros

    @pl.loop(0, rpt, step=zrows)
    def _(r):
        pltpu.sync_copy(zbuf, acc.at[pl.ds(t * rpt + r, zrows)])

    plsc.subcore_barrier()

    ebase0 = t * s_per

    def s1_issue(b, si):
        base = ebase0 + b * W
        pltpu.sync_copy(dst_hbm.at[pl.ds(base, W)], dstb.at[si])
        pltpu.sync_copy(src_hbm.at[pl.ds(base, W)], srcb.at[si])
        for j in range(W // LN):
            sl = pl.ds(j * LN, LN)
            idxq[si, sl] = dstb[si, sl] + cnp
            idxs[si, sl] = srcb[si, sl] + cnp
        pltpu.make_async_copy(qtab.at[idxq.at[si]], qbuf.at[si], semq[si]).start()
        pltpu.make_async_copy(ktab.at[idxs.at[si]], kbuf.at[si], semk[si]).start()

    def s1_compute(b, si):
        base = ebase0 + b * W
        pltpu.make_async_copy(qtab.at[idxq.at[si]], qbuf.at[si], semq[si]).wait()
        pltpu.make_async_copy(ktab.at[idxs.at[si]], kbuf.at[si], semk[si]).wait()

        @plsc.parallel_loop(0, W, unroll=4)
        def _(e):
            a = jnp.zeros((LN,), F32)
            for j in range(CH // (2 * LN)):
                sl = pl.ds(j * 2 * LN, 2 * LN)
                qa, qb = plsc.unpack(qbuf[si, e, sl],
                                     format=plsc.PackFormat.INTERLEAVED,
                                     preferred_element_type=F32)
                ka, kb = plsc.unpack(kbuf[si, e, sl],
                                     format=plsc.PackFormat.INTERLEAVED,
                                     preferred_element_type=F32)
                a = a + qa * ka + qb * kb
            plsc.store_scatter(albuf, [jnp.full((LN,), e, I32)],
                               plsc.cumsum(a), mask=m_last)

        for j in range(W // LN):
            sl = pl.ds(j * LN, LN)
            exb[si, sl] = jnp.exp(albuf[sl] * (1.0 / 16.0))
        pltpu.sync_copy(exb.at[si], ex_out.at[pl.ds(cep + base, W)])

        @plsc.parallel_loop(0, W, unroll=4)
        def _(e):
            wv = plsc.load_gather(exb.at[si], [jnp.full((LN,), e, I32)])
            for j in range(QW // LN):
                vbuf[si, e, pl.ds(j * LN, LN)] = wv

        pltpu.sync_copy(vbuf.at[si], acc.at[dstb.at[si]], add=True)

    s1_issue(0, 0)

    @pl.loop(0, nb, step=2)
    def _(b):
        s1_issue(b + 1, 1)
        s1_compute(b, 0)

        @pl.when(b + 2 < nb)
        def _():
            s1_issue(b + 2, 0)

        s1_compute(b + 1, 1)

    plsc.subcore_barrier()
    pltpu.sync_copy(acc.at[pl.ds(t * rpt, rpt)],
                    den_out.at[pl.ds(cnp + t * rpt, rpt)])

    for quarter in range(NQ):
        voff = (NQ * c + quarter) * np_

        @pl.loop(0, rpt, step=zrows)
        def _(r):
            pltpu.sync_copy(zbuf, acc.at[pl.ds(t * rpt + r, zrows)])

        plsc.subcore_barrier()

        def sw_issue(b, si):
            base = ebase0 + b * W
            pltpu.sync_copy(dst_hbm.at[pl.ds(base, W)], dstb.at[si])
            pltpu.sync_copy(src_hbm.at[pl.ds(base, W)], srcb.at[si])
            pltpu.sync_copy(ex_out.at[pl.ds(cep + base, W)], exb.at[si])
            for j in range(W // LN):
                sl = pl.ds(j * LN, LN)
                idxs[si, sl] = srcb[si, sl] + voff
            pltpu.make_async_copy(vtab.at[idxs.at[si]], vbuf.at[si],
                                  semq[si]).start()

        def sw_compute(b, si):
            pltpu.make_async_copy(vtab.at[idxs.at[si]], vbuf.at[si],
                                  semq[si]).wait()

            @plsc.parallel_loop(0, W, unroll=4)
            def _(e):
                wv = plsc.load_gather(exb.at[si], [jnp.full((LN,), e, I32)])
                for j in range(QW // LN):
                    sl = pl.ds(j * LN, LN)
                    vbuf[si, e, sl] = vbuf[si, e, sl] * wv

            pltpu.sync_copy(vbuf.at[si], acc.at[dstb.at[si]], add=True)

        sw_issue(0, 0)

        @pl.loop(0, nb, step=2)
        def _(b):
            sw_issue(b + 1, 1)
            sw_compute(b, 0)

            @pl.when(b + 2 < nb)
            def _():
                sw_issue(b + 2, 0)

            sw_compute(b + 1, 1)

        plsc.subcore_barrier()
        pltpu.sync_copy(acc.at[pl.ds(t * rpt, rpt)],
                        msg_out.at[pl.ds(voff + t * rpt, rpt)])
        plsc.subcore_barrier()


@functools.lru_cache(maxsize=None)
def _sc_call(np_, s_per, nb):
    mesh = plsc.VectorSubcoreMesh(core_axis_name="c", subcore_axis_name="s")
    cp = pltpu.CompilerParams()
    for fld, val in (("needs_layout_passes", False),
                     ("use_tc_tiling_on_sc", False)):
        if fld in pltpu.CompilerParams.__dataclass_fields__:
            cp = dataclasses.replace(cp, **{fld: val})
    return pl.kernel(
        functools.partial(_sc_body, np_=np_, s_per=s_per, nb=nb),
        out_type=(
            jax.ShapeDtypeStruct((H * np_, QW), F32),
            jax.ShapeDtypeStruct((H * NQ * np_, QW), F32),
            jax.ShapeDtypeStruct((H * NSUB * s_per,), F32),
        ),
        mesh=mesh,
        compiler_params=cp,
        scratch_types=[
            pltpu.VMEM((2, W), I32),
            pltpu.VMEM((2, W), I32),
            pltpu.VMEM((2, W), I32),
            pltpu.VMEM((2, W), I32),
            pltpu.VMEM((2, W, CH), BF16),
            pltpu.VMEM((2, W, CH), BF16),
            pltpu.VMEM((2, W, QW), F32),
            pltpu.VMEM((2, W), F32),
            pltpu.VMEM((W,), F32),
            pltpu.VMEM((32, QW), F32),
            pltpu.SemaphoreType.DMA,
            pltpu.SemaphoreType.DMA,
            pltpu.SemaphoreType.DMA,
            pltpu.SemaphoreType.DMA,
            pltpu.VMEM_SHARED((np_, QW), F32),
        ],
    )


def _fin_body(msg_ref, den_ref, s_ref, o_ref, *, act):
    o = s_ref[...]
    ms = []
    for h in range(H):
        d = den_ref[h, :, 0:1] + 1e-16
        m = jnp.concatenate([msg_ref[h, qq] for qq in range(NQ)], axis=1) / d
        ms.append(m)
    o = o + sum(ms) * (1.0 / H)
    if act:
        o = jnp.where(o >= 0, o, 0.1 * o)
    o_ref[...] = o


@functools.lru_cache(maxsize=None)
def _fin_call(np_, bn, act):
    return pl.pallas_call(
        functools.partial(_fin_body, act=act),
        grid=(np_ // bn,),
        in_specs=[
            pl.BlockSpec((H, NQ, bn, QW), lambda i: (0, 0, i, 0)),
            pl.BlockSpec((H, bn, QW), lambda i: (0, i, 0)),
            pl.BlockSpec((bn, CH), lambda i: (i, 0)),
        ],
        out_specs=pl.BlockSpec((bn, CH), lambda i: (i, 0)),
        out_shape=jax.ShapeDtypeStruct((np_, CH), F32),
    )


def _layer(xp, dstp, srcp, wq, bq, wk, bk, wv, bv, ws, bs,
           np_, s_per, nb, n_valid, act):
    q, k, v, s = _qkv_call(np_, xp.shape[1], n_valid, 512)(
        xp, wq, bq.reshape(1, -1), wk, bk.reshape(1, -1),
        wv, bv.reshape(1, -1), ws, bs.reshape(1, -1))
    den, msg, _ = _sc_call(np_, s_per, nb)(
        q.reshape(H * np_, CH), k.reshape(H * np_, CH),
        v.reshape(H * NQ * np_, QW), dstp, srcp)
    return _fin_call(np_, 512, act)(
        msg.reshape(H, NQ, np_, QW), den.reshape(H, np_, QW), s)


def kernel(x, edge_index_list, Wq1, bq1, Wk1, bk1, Wv1, bv1, Ws1, bs1,
           Wq2, bq2, Wk2, bk2, Wv2, bv2, Ws2, bs2):
    B, N, Cin = x.shape
    E = B * edge_index_list.shape[2]
    offs = (jnp.arange(B, dtype=edge_index_list.dtype) * N)[:, None, None]
    flat = jnp.transpose(edge_index_list + offs, (1, 0, 2)).reshape(2, -1)
    src = flat[0].astype(I32)
    dst = flat[1].astype(I32)

    np_ = ((B * N + 2048) // 2048) * 2048
    s_per = ((E + NSUB * 2 * W - 1) // (NSUB * 2 * W)) * 2 * W
    nb = s_per // W
    ep = NSUB * s_per
    pad = jnp.full((ep - E,), B * N, I32)
    srcp = jnp.concatenate([src, pad])
    dstp = jnp.concatenate([dst, pad])
    xp = jnp.pad(x.reshape(B * N, Cin), ((0, np_ - B * N), (0, 0)))

    h = _layer(xp, dstp, srcp, Wq1, bq1, Wk1, bk1, Wv1, bv1, Ws1, bs1,
               np_, s_per, nb, B * N, act=True)
    o = _layer(h, dstp, srcp, Wq2, bq2, Wk2, bk2, Wv2, bv2, Ws2, bs2,
               np_, s_per, nb, B * N, act=False)
    return o[:B * N].reshape(B, N, CH)


# speedup vs baseline: 13.9588x; 1.0006x over previous
import dataclasses
import functools

import jax
import jax.numpy as jnp
from jax import lax
from jax.experimental import pallas as pl
from jax.experimental.pallas import tpu as pltpu
from jax.experimental.pallas import tpu_sc as plsc

F32 = jnp.float32
BF16 = jnp.bfloat16
I32 = jnp.int32
LN = 16
H = 2
CH = 256
QW = 64
NQ = CH // QW
W = 128
NSUB = 16


def _qkv_body(x_ref, wq_ref, bq_ref, wk_ref, bk_ref, wv_ref, bv_ref,
              ws_ref, bs_ref, q_ref, k_ref, v_ref, s_ref, *, bn, n_valid):
    i = pl.program_id(0)
    rows = i * bn + lax.broadcasted_iota(I32, (bn, 1), 0)
    m = (rows < n_valid).astype(F32)
    x = x_ref[...] * m
    q = (jnp.dot(x, wq_ref[...], preferred_element_type=F32) + bq_ref[...]) * m
    k = (jnp.dot(x, wk_ref[...], preferred_element_type=F32) + bk_ref[...]) * m
    v = (jnp.dot(x, wv_ref[...], preferred_element_type=F32) + bv_ref[...]) * m
    s = (jnp.dot(x, ws_ref[...], preferred_element_type=F32) + bs_ref[...]) * m
    q16, k16 = q.astype(BF16), k.astype(BF16)
    q_ref[0], q_ref[1] = q16[:, :CH], q16[:, CH:]
    k_ref[0], k_ref[1] = k16[:, :CH], k16[:, CH:]
    for qq in range(H * NQ):
        v_ref[qq] = v[:, qq * QW:(qq + 1) * QW]
    s_ref[...] = s


@functools.lru_cache(maxsize=None)
def _qkv_call(np_, cin, n_valid, bn):
    return pl.pallas_call(
        functools.partial(_qkv_body, bn=bn, n_valid=n_valid),
        grid=(np_ // bn,),
        in_specs=[
            pl.BlockSpec((bn, cin), lambda i: (i, 0)),
            pl.BlockSpec((cin, H * CH), lambda i: (0, 0)),
            pl.BlockSpec((1, H * CH), lambda i: (0, 0)),
            pl.BlockSpec((cin, H * CH), lambda i: (0, 0)),
            pl.BlockSpec((1, H * CH), lambda i: (0, 0)),
            pl.BlockSpec((cin, H * CH), lambda i: (0, 0)),
            pl.BlockSpec((1, H * CH), lambda i: (0, 0)),
            pl.BlockSpec((cin, CH), lambda i: (0, 0)),
            pl.BlockSpec((1, CH), lambda i: (0, 0)),
        ],
        out_specs=[
            pl.BlockSpec((H, bn, CH), lambda i: (0, i, 0)),
            pl.BlockSpec((H, bn, CH), lambda i: (0, i, 0)),
            pl.BlockSpec((H * NQ, bn, QW), lambda i: (0, i, 0)),
            pl.BlockSpec((bn, CH), lambda i: (i, 0)),
        ],
        out_shape=[
            jax.ShapeDtypeStruct((H, np_, CH), BF16),
            jax.ShapeDtypeStruct((H, np_, CH), BF16),
            jax.ShapeDtypeStruct((H * NQ, np_, QW), F32),
            jax.ShapeDtypeStruct((np_, CH), F32),
        ],
    )


def _sc_body(qtab, ktab, vtab, dst_hbm, src_hbm, den_out, msg_out, ex_out,
             dstb, srcb, idxq, idxs, qbuf, kbuf, vbuf, exb, albuf,
             zbuf, sq0, sq1, sk0, sk1, acc, *, np_, s_per, nb):
    c = lax.axis_index("c")
    t = lax.axis_index("s")
    ep = NSUB * s_per
    rpt = np_ // NSUB
    cnp = c * np_
    cep = c * ep
    lane = lax.iota(I32, LN)
    m_last = lane == (LN - 1)
    zeros = jnp.zeros((LN,), F32)
    zrows = zbuf.shape[0]
    semq = (sq0, sq1)
    semk = (sk0, sk1)

    @pl.loop(0, zrows)
    def _(r):
        for j in range(QW // LN):
            zbuf[r, pl.ds(j * LN, LN)] = zeros

    @pl.loop(0, rpt, step=zrows)
    def _(r):
        pltpu.sync_copy(zbuf, acc.at[pl.ds(t * rpt + r, zrows)])

    plsc.subcore_barrier()

    ebase0 = t * s_per

    def s1_issue(b, si):
        base = ebase0 + b * W
        pltpu.sync_copy(dst_hbm.at[pl.ds(base, W)], dstb.at[si])
        pltpu.sync_copy(src_hbm.at[pl.ds(base, W)], srcb.at[si])
        for j in range(W // LN):
            sl = pl.ds(j * LN, LN)
            idxq[si, sl] = dstb[si, sl] + cnp
            idxs[si, sl] = srcb[si, sl] + cnp
        pltpu.make_async_copy(qtab.at[idxq.at[si]], qbuf.at[si], semq[si]).start()
        pltpu.make_async_copy(ktab.at[idxs.at[si]], kbuf.at[si], semk[si]).start()

    def s1_compute(b, si):
        base = ebase0 + b * W
        pltpu.make_async_copy(qtab.at[idxq.at[si]], qbuf.at[si], semq[si]).wait()
        pltpu.make_async_copy(ktab.at[idxs.at[si]], kbuf.at[si], semk[si]).wait()

        @plsc.parallel_loop(0, W, unroll=4)
        def _(e):
            a = jnp.zeros((LN,), F32)
            for j in range(CH // (2 * LN)):
                sl = pl.ds(j * 2 * LN, 2 * LN)
                qa, qb = plsc.unpack(qbuf[si, e, sl],
                                     format=plsc.PackFormat.INTERLEAVED,
                                     preferred_element_type=F32)
                ka, kb = plsc.unpack(kbuf[si, e, sl],
                                     format=plsc.PackFormat.INTERLEAVED,
                                     preferred_element_type=F32)
                a = a + qa * ka + qb * kb
            plsc.store_scatter(albuf, [jnp.full((LN,), e, I32)],
                               plsc.cumsum(a), mask=m_last)

        for j in range(W // LN):
            sl = pl.ds(j * LN, LN)
            exb[si, sl] = jnp.exp(albuf[sl] * (1.0 / 16.0))
        pltpu.sync_copy(exb.at[si], ex_out.at[pl.ds(cep + base, W)])

        @plsc.parallel_loop(0, W, unroll=4)
        def _(e):
            wv = plsc.load_gather(exb.at[si], [jnp.full((LN,), e, I32)])
            for j in range(QW // LN):
                vbuf[si, e, pl.ds(j * LN, LN)] = wv

        pltpu.sync_copy(vbuf.at[si], acc.at[dstb.at[si]], add=True)

    with jax.named_scope("s1"):
        s1_issue(0, 0)

        @pl.loop(0, nb, step=2)
        def _(b):
            s1_issue(b + 1, 1)
            s1_compute(b, 0)

            @pl.when(b + 2 < nb)
            def _():
                s1_issue(b + 2, 0)

            s1_compute(b + 1, 1)

    plsc.subcore_barrier()
    pltpu.sync_copy(acc.at[pl.ds(t * rpt, rpt)],
                    den_out.at[pl.ds(cnp + t * rpt, rpt)])

    for quarter in range(NQ):
        voff = (NQ * c + quarter) * np_

        @pl.loop(0, rpt, step=zrows)
        def _(r):
            pltpu.sync_copy(zbuf, acc.at[pl.ds(t * rpt + r, zrows)])

        plsc.subcore_barrier()

        def sw_issue(b, si):
            base = ebase0 + b * W
            pltpu.sync_copy(dst_hbm.at[pl.ds(base, W)], dstb.at[si])
            pltpu.sync_copy(src_hbm.at[pl.ds(base, W)], srcb.at[si])
            pltpu.sync_copy(ex_out.at[pl.ds(cep + base, W)], exb.at[si])
            for j in range(W // LN):
                sl = pl.ds(j * LN, LN)
                idxs[si, sl] = srcb[si, sl] + voff
            pltpu.make_async_copy(vtab.at[idxs.at[si]], vbuf.at[si],
                                  semq[si]).start()

        def sw_compute(b, si):
            pltpu.make_async_copy(vtab.at[idxs.at[si]], vbuf.at[si],
                                  semq[si]).wait()

            @plsc.parallel_loop(0, W, unroll=4)
            def _(e):
                wv = plsc.load_gather(exb.at[si], [jnp.full((LN,), e, I32)])
                for j in range(QW // LN):
                    sl = pl.ds(j * LN, LN)
                    vbuf[si, e, sl] = vbuf[si, e, sl] * wv

            pltpu.sync_copy(vbuf.at[si], acc.at[dstb.at[si]], add=True)

        with jax.named_scope(f"sw{quarter}"):
            sw_issue(0, 0)

            @pl.loop(0, nb, step=2)
            def _(b):
                sw_issue(b + 1, 1)
                sw_compute(b, 0)

                @pl.when(b + 2 < nb)
                def _():
                    sw_issue(b + 2, 0)

                sw_compute(b + 1, 1)

        plsc.subcore_barrier()
        pltpu.sync_copy(acc.at[pl.ds(t * rpt, rpt)],
                        msg_out.at[pl.ds(voff + t * rpt, rpt)])
        plsc.subcore_barrier()


@functools.lru_cache(maxsize=None)
def _sc_call(np_, s_per, nb):
    mesh = plsc.VectorSubcoreMesh(core_axis_name="c", subcore_axis_name="s")
    cp = pltpu.CompilerParams()
    for fld, val in (("needs_layout_passes", False),
                     ("use_tc_tiling_on_sc", False)):
        if fld in pltpu.CompilerParams.__dataclass_fields__:
            cp = dataclasses.replace(cp, **{fld: val})
    return pl.kernel(
        functools.partial(_sc_body, np_=np_, s_per=s_per, nb=nb),
        out_type=(
            jax.ShapeDtypeStruct((H * np_, QW), F32),
            jax.ShapeDtypeStruct((H * NQ * np_, QW), F32),
            jax.ShapeDtypeStruct((H * NSUB * s_per,), F32),
        ),
        mesh=mesh,
        compiler_params=cp,
        scratch_types=[
            pltpu.VMEM((2, W), I32),
            pltpu.VMEM((2, W), I32),
            pltpu.VMEM((2, W), I32),
            pltpu.VMEM((2, W), I32),
            pltpu.VMEM((2, W, CH), BF16),
            pltpu.VMEM((2, W, CH), BF16),
            pltpu.VMEM((2, W, QW), F32),
            pltpu.VMEM((2, W), F32),
            pltpu.VMEM((W,), F32),
            pltpu.VMEM((32, QW), F32),
            pltpu.SemaphoreType.DMA,
            pltpu.SemaphoreType.DMA,
            pltpu.SemaphoreType.DMA,
            pltpu.SemaphoreType.DMA,
            pltpu.VMEM_SHARED((np_, QW), F32),
        ],
    )


def _fin_body(msg_ref, den_ref, s_ref, o_ref, *, act):
    o = s_ref[...]
    ms = []
    for h in range(H):
        d = den_ref[h, :, 0:1] + 1e-16
        m = jnp.concatenate([msg_ref[h, qq] for qq in range(NQ)], axis=1) / d
        ms.append(m)
    o = o + sum(ms) * (1.0 / H)
    if act:
        o = jnp.where(o >= 0, o, 0.1 * o)
    o_ref[...] = o


@functools.lru_cache(maxsize=None)
def _fin_call(np_, bn, act):
    return pl.pallas_call(
        functools.partial(_fin_body, act=act),
        grid=(np_ // bn,),
        in_specs=[
            pl.BlockSpec((H, NQ, bn, QW), lambda i: (0, 0, i, 0)),
            pl.BlockSpec((H, bn, QW), lambda i: (0, i, 0)),
            pl.BlockSpec((bn, CH), lambda i: (i, 0)),
        ],
        out_specs=pl.BlockSpec((bn, CH), lambda i: (i, 0)),
        out_shape=jax.ShapeDtypeStruct((np_, CH), F32),
    )


def _layer(xp, dstp, srcp, wq, bq, wk, bk, wv, bv, ws, bs,
           np_, s_per, nb, n_valid, act):
    q, k, v, s = _qkv_call(np_, xp.shape[1], n_valid, 512)(
        xp, wq, bq.reshape(1, -1), wk, bk.reshape(1, -1),
        wv, bv.reshape(1, -1), ws, bs.reshape(1, -1))
    den, msg, _ = _sc_call(np_, s_per, nb)(
        q.reshape(H * np_, CH), k.reshape(H * np_, CH),
        v.reshape(H * NQ * np_, QW), dstp, srcp)
    return _fin_call(np_, 512, act)(
        msg.reshape(H, NQ, np_, QW), den.reshape(H, np_, QW), s)


def kernel(x, edge_index_list, Wq1, bq1, Wk1, bk1, Wv1, bv1, Ws1, bs1,
           Wq2, bq2, Wk2, bk2, Wv2, bv2, Ws2, bs2):
    B, N, Cin = x.shape
    E = B * edge_index_list.shape[2]
    offs = (jnp.arange(B, dtype=edge_index_list.dtype) * N)[:, None, None]
    flat = jnp.transpose(edge_index_list + offs, (1, 0, 2)).reshape(2, -1)
    src = flat[0].astype(I32)
    dst = flat[1].astype(I32)

    np_ = ((B * N + 2048) // 2048) * 2048
    s_per = ((E + NSUB * 2 * W - 1) // (NSUB * 2 * W)) * 2 * W
    nb = s_per // W
    ep = NSUB * s_per
    pad = jnp.full((ep - E,), B * N, I32)
    srcp = jnp.concatenate([src, pad])
    dstp = jnp.concatenate([dst, pad])
    xp = jnp.pad(x.reshape(B * N, Cin), ((0, np_ - B * N), (0, 0)))

    h = _layer(xp, dstp, srcp, Wq1, bq1, Wk1, bk1, Wv1, bv1, Ws1, bs1,
               np_, s_per, nb, B * N, act=True)
    o = _layer(h, dstp, srcp, Wq2, bq2, Wk2, bk2, Wv2, bv2, Ws2, bs2,
               np_, s_per, nb, B * N, act=False)
    return o[:B * N].reshape(B, N, CH)


# speedup vs baseline: 18.2993x; 1.3109x over previous
import dataclasses
import functools

import jax
import jax.numpy as jnp
from jax import lax
from jax.experimental import pallas as pl
from jax.experimental.pallas import tpu as pltpu
from jax.experimental.pallas import tpu_sc as plsc

F32 = jnp.float32
BF16 = jnp.bfloat16
I32 = jnp.int32
LN = 16
H = 2
CH = 256
SW = 128
NS = CH // SW
W = 64
NSUB = 16
BN = 1264


def _qkv_body(x_ref, wq_ref, bq_ref, wk_ref, bk_ref, wv_ref, bv_ref,
              ws_ref, bs_ref, q_ref, k_ref, v_ref, s_ref, *, bn, n_valid):
    i = pl.program_id(0)
    rows = i * bn + lax.broadcasted_iota(I32, (bn, 1), 0)
    m = (rows < n_valid).astype(F32)
    x = x_ref[...] * m
    q = (jnp.dot(x, wq_ref[...], preferred_element_type=F32) + bq_ref[...]) * m
    k = (jnp.dot(x, wk_ref[...], preferred_element_type=F32) + bk_ref[...]) * m
    v = (jnp.dot(x, wv_ref[...], preferred_element_type=F32) + bv_ref[...]) * m
    s = (jnp.dot(x, ws_ref[...], preferred_element_type=F32) + bs_ref[...]) * m
    q16, k16 = q.astype(BF16), k.astype(BF16)
    q_ref[0], q_ref[1] = q16[:, :CH], q16[:, CH:]
    k_ref[0], k_ref[1] = k16[:, :CH], k16[:, CH:]
    for j in range(H * NS):
        v_ref[j] = v[:, j * SW:(j + 1) * SW]
    s_ref[...] = s


@functools.lru_cache(maxsize=None)
def _qkv_call(np_, cin, n_valid, bn):
    return pl.pallas_call(
        functools.partial(_qkv_body, bn=bn, n_valid=n_valid),
        grid=(np_ // bn,),
        in_specs=[
            pl.BlockSpec((bn, cin), lambda i: (i, 0)),
            pl.BlockSpec((cin, H * CH), lambda i: (0, 0)),
            pl.BlockSpec((1, H * CH), lambda i: (0, 0)),
            pl.BlockSpec((cin, H * CH), lambda i: (0, 0)),
            pl.BlockSpec((1, H * CH), lambda i: (0, 0)),
            pl.BlockSpec((cin, H * CH), lambda i: (0, 0)),
            pl.BlockSpec((1, H * CH), lambda i: (0, 0)),
            pl.BlockSpec((cin, CH), lambda i: (0, 0)),
            pl.BlockSpec((1, CH), lambda i: (0, 0)),
        ],
        out_specs=[
            pl.BlockSpec((H, bn, CH), lambda i: (0, i, 0)),
            pl.BlockSpec((H, bn, CH), lambda i: (0, i, 0)),
            pl.BlockSpec((H * NS, bn, SW), lambda i: (0, i, 0)),
            pl.BlockSpec((bn, CH), lambda i: (i, 0)),
        ],
        out_shape=[
            jax.ShapeDtypeStruct((H, np_, CH), BF16),
            jax.ShapeDtypeStruct((H, np_, CH), BF16),
            jax.ShapeDtypeStruct((H * NS, np_, SW), F32),
            jax.ShapeDtypeStruct((np_, CH), F32),
        ],
    )


def _sc_body(qtab, ktab, vtab, dst_hbm, src_hbm, zin, den_out, msg_out, ex_out,
             dstb, srcb, gq, gk, qbuf, kbuf, vbuf, exb,
             sr0, sr1, sr2, sr3, sq0, sq1, sk0, sk1, se0, se1,
             acc, *, np_, s_per, nb):
    c = lax.axis_index("c")
    t = lax.axis_index("s")
    ep = NSUB * s_per
    rpt = np_ // NSUB
    cnp = c * np_
    cep = c * ep
    lane = lax.iota(I32, LN)
    m_last = lane == (LN - 1)
    sr = (sr0, sr1, sr2, sr3)
    sq = (sq0, sq1)
    sk = (sk0, sk1)
    se = (se0, se1)
    ebase0 = t * s_per

    def zero_acc():
        pltpu.sync_copy(zin, acc.at[pl.ds(t * rpt, rpt)])

    def s1_raw(b, d):
        base = ebase0 + b * W
        pltpu.make_async_copy(dst_hbm.at[pl.ds(base, W)], dstb.at[d],
                              sr[d]).start()
        pltpu.make_async_copy(src_hbm.at[pl.ds(base, W)], srcb.at[d],
                              sr[d]).start()

    def s1_gather(b, d, si):
        base = ebase0 + b * W
        pltpu.make_async_copy(dst_hbm.at[pl.ds(base, W)], dstb.at[d],
                              sr[d]).wait()
        pltpu.make_async_copy(src_hbm.at[pl.ds(base, W)], srcb.at[d],
                              sr[d]).wait()
        for j in range(W // LN):
            sl = pl.ds(j * LN, LN)
            gq[si, sl] = dstb[d, sl] + cnp
            gk[si, sl] = srcb[d, sl] + cnp
        pltpu.make_async_copy(qtab.at[gq.at[si]], qbuf.at[si], sq[si]).start()
        pltpu.make_async_copy(ktab.at[gk.at[si]], kbuf.at[si], sk[si]).start()

    def s1_compute(b, d, si):
        base = ebase0 + b * W
        pltpu.make_async_copy(qtab.at[gq.at[si]], qbuf.at[si], sq[si]).wait()
        pltpu.make_async_copy(ktab.at[gk.at[si]], kbuf.at[si], sk[si]).wait()

        @plsc.parallel_loop(0, W, unroll=4)
        def _(e):
            a = jnp.zeros((LN,), F32)
            for j in range(CH // (2 * LN)):
                sl = pl.ds(j * 2 * LN, 2 * LN)
                qa, qb = plsc.unpack(qbuf[si, e, sl],
                                     format=plsc.PackFormat.INTERLEAVED,
                                     preferred_element_type=F32)
                ka, kb = plsc.unpack(kbuf[si, e, sl],
                                     format=plsc.PackFormat.INTERLEAVED,
                                     preferred_element_type=F32)
                a = a + qa * ka + qb * kb
            plsc.store_scatter(exb.at[si], [jnp.full((LN,), e, I32)],
                               plsc.cumsum(a), mask=m_last)

        for j in range(W // LN):
            sl = pl.ds(j * LN, LN)
            exb[si, sl] = jnp.exp(exb[si, sl] * (1.0 / 16.0))
        pltpu.sync_copy(exb.at[si], ex_out.at[pl.ds(cep + base, W)])

        @plsc.parallel_loop(0, W, unroll=4)
        def _(e):
            wv = plsc.load_gather(exb.at[si], [jnp.full((LN,), e, I32)])
            for j in range(SW // LN):
                vbuf[si, e, pl.ds(j * LN, LN)] = wv

        pltpu.sync_copy(vbuf.at[si], acc.at[dstb.at[d]], add=True)

    zero_acc()
    plsc.subcore_barrier()

    for u in range(4):
        s1_raw(u, u)
    s1_gather(0, 0, 0)

    @pl.loop(0, nb, step=4)
    def _(b):
        for u in range(4):
            @pl.when(b + u + 1 < nb)
            def _():
                s1_gather(b + u + 1, (u + 1) % 4, (u + 1) % 2)

            s1_compute(b + u, u, u % 2)

            @pl.when(b + u + 4 < nb)
            def _():
                s1_raw(b + u + 4, u)

    plsc.subcore_barrier()
    pltpu.sync_copy(acc.at[pl.ds(t * rpt, rpt)],
                    den_out.at[pl.ds(cnp + t * rpt, rpt)])

    for half in range(NS):
        voff = (NS * c + half) * np_

        zero_acc()
        plsc.subcore_barrier()

        def sw_raw(b, d):
            base = ebase0 + b * W
            pltpu.make_async_copy(dst_hbm.at[pl.ds(base, W)], dstb.at[d],
                                  sr[d]).start()
            pltpu.make_async_copy(src_hbm.at[pl.ds(base, W)], srcb.at[d],
                                  sr[d]).start()

        def sw_gather(b, d, si):
            base = ebase0 + b * W
            pltpu.make_async_copy(dst_hbm.at[pl.ds(base, W)], dstb.at[d],
                                  sr[d]).wait()
            pltpu.make_async_copy(src_hbm.at[pl.ds(base, W)], srcb.at[d],
                                  sr[d]).wait()
            for j in range(W // LN):
                sl = pl.ds(j * LN, LN)
                gq[si, sl] = srcb[d, sl] + voff
            pltpu.make_async_copy(vtab.at[gq.at[si]], vbuf.at[si],
                                  sq[si]).start()
            pltpu.make_async_copy(ex_out.at[pl.ds(cep + base, W)], exb.at[si],
                                  se[si]).start()

        def sw_compute(b, d, si):
            base = ebase0 + b * W
            pltpu.make_async_copy(vtab.at[gq.at[si]], vbuf.at[si],
                                  sq[si]).wait()
            pltpu.make_async_copy(ex_out.at[pl.ds(cep + base, W)], exb.at[si],
                                  se[si]).wait()

            @plsc.parallel_loop(0, W, unroll=4)
            def _(e):
                wv = plsc.load_gather(exb.at[si], [jnp.full((LN,), e, I32)])
                for j in range(SW // LN):
                    sl = pl.ds(j * LN, LN)
                    vbuf[si, e, sl] = vbuf[si, e, sl] * wv

            pltpu.sync_copy(vbuf.at[si], acc.at[dstb.at[d]], add=True)

        for u in range(4):
            sw_raw(u, u)
        sw_gather(0, 0, 0)

        @pl.loop(0, nb, step=4)
        def _(b):
            for u in range(4):
                @pl.when(b + u + 1 < nb)
                def _():
                    sw_gather(b + u + 1, (u + 1) % 4, (u + 1) % 2)

                sw_compute(b + u, u, u % 2)

                @pl.when(b + u + 4 < nb)
                def _():
                    sw_raw(b + u + 4, u)

        plsc.subcore_barrier()
        pltpu.sync_copy(acc.at[pl.ds(t * rpt, rpt)],
                        msg_out.at[pl.ds(voff + t * rpt, rpt)])
        plsc.subcore_barrier()


@functools.lru_cache(maxsize=None)
def _sc_call(np_, s_per, nb):
    mesh = plsc.VectorSubcoreMesh(core_axis_name="c", subcore_axis_name="s")
    cp = pltpu.CompilerParams()
    for fld, val in (("needs_layout_passes", False),
                     ("use_tc_tiling_on_sc", False)):
        if fld in pltpu.CompilerParams.__dataclass_fields__:
            cp = dataclasses.replace(cp, **{fld: val})
    return pl.kernel(
        functools.partial(_sc_body, np_=np_, s_per=s_per, nb=nb),
        out_type=(
            jax.ShapeDtypeStruct((H * np_, SW), F32),
            jax.ShapeDtypeStruct((H * NS * np_, SW), F32),
            jax.ShapeDtypeStruct((H * NSUB * s_per,), F32),
        ),
        mesh=mesh,
        compiler_params=cp,
        scratch_types=[
            pltpu.VMEM((4, W), I32),
            pltpu.VMEM((4, W), I32),
            pltpu.VMEM((2, W), I32),
            pltpu.VMEM((2, W), I32),
            pltpu.VMEM((2, W, CH), BF16),
            pltpu.VMEM((2, W, CH), BF16),
            pltpu.VMEM((2, W, SW), F32),
            pltpu.VMEM((2, W), F32),
            pltpu.SemaphoreType.DMA,
            pltpu.SemaphoreType.DMA,
            pltpu.SemaphoreType.DMA,
            pltpu.SemaphoreType.DMA,
            pltpu.SemaphoreType.DMA,
            pltpu.SemaphoreType.DMA,
            pltpu.SemaphoreType.DMA,
            pltpu.SemaphoreType.DMA,
            pltpu.SemaphoreType.DMA,
            pltpu.SemaphoreType.DMA,
            pltpu.VMEM_SHARED((np_, SW), F32),
        ],
    )


def _fin_body(msg_ref, den_ref, s_ref, o_ref, *, act):
    o = s_ref[...]
    ms = []
    for h in range(H):
        d = den_ref[h, :, 0:1] + 1e-16
        m = jnp.concatenate([msg_ref[h, j] for j in range(NS)], axis=1) / d
        ms.append(m)
    o = o + sum(ms) * (1.0 / H)
    if act:
        o = jnp.where(o >= 0, o, 0.1 * o)
    o_ref[...] = o


@functools.lru_cache(maxsize=None)
def _fin_call(np_, bn, act):
    return pl.pallas_call(
        functools.partial(_fin_body, act=act),
        grid=(np_ // bn,),
        in_specs=[
            pl.BlockSpec((H, NS, bn, SW), lambda i: (0, 0, i, 0)),
            pl.BlockSpec((H, bn, SW), lambda i: (0, i, 0)),
            pl.BlockSpec((bn, CH), lambda i: (i, 0)),
        ],
        out_specs=pl.BlockSpec((bn, CH), lambda i: (i, 0)),
        out_shape=jax.ShapeDtypeStruct((np_, CH), F32),
    )


def _layer(xp, dstp, srcp, zin, wq, bq, wk, bk, wv, bv, ws, bs,
           np_, s_per, nb, n_valid, act):
    q, k, v, s = _qkv_call(np_, xp.shape[1], n_valid, BN)(
        xp, wq, bq.reshape(1, -1), wk, bk.reshape(1, -1),
        wv, bv.reshape(1, -1), ws, bs.reshape(1, -1))
    den, msg, _ = _sc_call(np_, s_per, nb)(
        q.reshape(H * np_, CH), k.reshape(H * np_, CH),
        v.reshape(H * NS * np_, SW), dstp, srcp, zin)
    return _fin_call(np_, BN, act)(
        msg.reshape(H, NS, np_, SW), den.reshape(H, np_, SW), s)


def kernel(x, edge_index_list, Wq1, bq1, Wk1, bk1, Wv1, bv1, Ws1, bs1,
           Wq2, bq2, Wk2, bk2, Wv2, bv2, Ws2, bs2):
    B, N, Cin = x.shape
    E = B * edge_index_list.shape[2]
    offs = (jnp.arange(B, dtype=edge_index_list.dtype) * N)[:, None, None]
    flat = jnp.transpose(edge_index_list + offs, (1, 0, 2)).reshape(2, -1)
    src = flat[0].astype(I32)
    dst = flat[1].astype(I32)

    np_ = ((B * N + BN) // BN) * BN
    s_per = ((E + NSUB * 4 * W - 1) // (NSUB * 4 * W)) * 4 * W
    nb = s_per // W
    ep = NSUB * s_per
    pad = jnp.full((ep - E,), B * N, I32)
    srcp = jnp.concatenate([src, pad])
    dstp = jnp.concatenate([dst, pad])
    xp = jnp.pad(x.reshape(B * N, Cin), ((0, np_ - B * N), (0, 0)))
    zin = jnp.zeros((np_ // NSUB, SW), F32)

    h = _layer(xp, dstp, srcp, zin, Wq1, bq1, Wk1, bk1, Wv1, bv1, Ws1, bs1,
               np_, s_per, nb, B * N, act=True)
    o = _layer(h, dstp, srcp, zin, Wq2, bq2, Wk2, bk2, Wv2, bv2, Ws2, bs2,
               np_, s_per, nb, B * N, act=False)
    return o[:B * N].reshape(B, N, CH)


# speedup vs baseline: 18.6995x; 1.0219x over previous
import dataclasses
import functools

import jax
import jax.numpy as jnp
from jax import lax
from jax.experimental import pallas as pl
from jax.experimental.pallas import tpu as pltpu
from jax.experimental.pallas import tpu_sc as plsc

F32 = jnp.float32
BF16 = jnp.bfloat16
I32 = jnp.int32
LN = 16
H = 2
CH = 256
SW = 128
NS = CH // SW
W = 64
NSUB = 16
BN = 1264


def _qkv_body(x_ref, wq_ref, bq_ref, wk_ref, bk_ref, wv_ref, bv_ref,
              ws_ref, bs_ref, q_ref, k_ref, v_ref, s_ref, *, bn, n_valid):
    i = pl.program_id(0)
    rows = i * bn + lax.broadcasted_iota(I32, (bn, 1), 0)
    m = (rows < n_valid).astype(F32)
    x = x_ref[...] * m
    q = (jnp.dot(x, wq_ref[...], preferred_element_type=F32) + bq_ref[...]) * m
    k = (jnp.dot(x, wk_ref[...], preferred_element_type=F32) + bk_ref[...]) * m
    v = (jnp.dot(x, wv_ref[...], preferred_element_type=F32) + bv_ref[...]) * m
    s = (jnp.dot(x, ws_ref[...], preferred_element_type=F32) + bs_ref[...]) * m
    q16, k16 = q.astype(BF16), k.astype(BF16)
    q_ref[0], q_ref[1] = q16[:, :CH], q16[:, CH:]
    k_ref[0], k_ref[1] = k16[:, :CH], k16[:, CH:]
    for j in range(H * NS):
        v_ref[j] = v[:, j * SW:(j + 1) * SW]
    s_ref[...] = s


@functools.lru_cache(maxsize=None)
def _qkv_call(np_, cin, n_valid, bn):
    return pl.pallas_call(
        functools.partial(_qkv_body, bn=bn, n_valid=n_valid),
        grid=(np_ // bn,),
        in_specs=[
            pl.BlockSpec((bn, cin), lambda i: (i, 0)),
            pl.BlockSpec((cin, H * CH), lambda i: (0, 0)),
            pl.BlockSpec((1, H * CH), lambda i: (0, 0)),
            pl.BlockSpec((cin, H * CH), lambda i: (0, 0)),
            pl.BlockSpec((1, H * CH), lambda i: (0, 0)),
            pl.BlockSpec((cin, H * CH), lambda i: (0, 0)),
            pl.BlockSpec((1, H * CH), lambda i: (0, 0)),
            pl.BlockSpec((cin, CH), lambda i: (0, 0)),
            pl.BlockSpec((1, CH), lambda i: (0, 0)),
        ],
        out_specs=[
            pl.BlockSpec((H, bn, CH), lambda i: (0, i, 0)),
            pl.BlockSpec((H, bn, CH), lambda i: (0, i, 0)),
            pl.BlockSpec((H * NS, bn, SW), lambda i: (0, i, 0)),
            pl.BlockSpec((bn, CH), lambda i: (i, 0)),
        ],
        out_shape=[
            jax.ShapeDtypeStruct((H, np_, CH), BF16),
            jax.ShapeDtypeStruct((H, np_, CH), BF16),
            jax.ShapeDtypeStruct((H * NS, np_, SW), F32),
            jax.ShapeDtypeStruct((np_, CH), F32),
        ],
    )


def _sc_body(qtab, ktab, vtab, dst_hbm, src_hbm, zin, den_out, msg_out, ex_out,
             dstb, srcb, gq, gk, qbuf, kbuf, vbuf, exb,
             sr0, sr1, sr2, sr3, sq0, sq1, sk0, sk1, se0, se1, sa0, sa1,
             acc, *, np_, s_per, nb):
    c = lax.axis_index("c")
    t = lax.axis_index("s")
    ep = NSUB * s_per
    rpt = np_ // NSUB
    cnp = c * np_
    cep = c * ep
    lane = lax.iota(I32, LN)
    m_last = lane == (LN - 1)
    sr = (sr0, sr1, sr2, sr3)
    sq = (sq0, sq1)
    sk = (sk0, sk1)
    se = (se0, se1)
    sa = (sa0, sa1)
    ebase0 = t * s_per

    def zero_acc():
        pltpu.sync_copy(zin, acc.at[pl.ds(t * rpt, rpt)])

    def s1_raw(b, d):
        base = ebase0 + b * W
        pltpu.make_async_copy(dst_hbm.at[pl.ds(base, W)], dstb.at[d],
                              sr[d]).start()
        pltpu.make_async_copy(src_hbm.at[pl.ds(base, W)], srcb.at[d],
                              sr[d]).start()

    def s1_gather(b, d, si):
        base = ebase0 + b * W
        pltpu.make_async_copy(dst_hbm.at[pl.ds(base, W)], dstb.at[d],
                              sr[d]).wait()
        pltpu.make_async_copy(src_hbm.at[pl.ds(base, W)], srcb.at[d],
                              sr[d]).wait()
        for j in range(W // LN):
            sl = pl.ds(j * LN, LN)
            gq[si, sl] = dstb[d, sl] + cnp
            gk[si, sl] = srcb[d, sl] + cnp
        pltpu.make_async_copy(qtab.at[gq.at[si]], qbuf.at[si], sq[si]).start()
        pltpu.make_async_copy(ktab.at[gk.at[si]], kbuf.at[si], sk[si]).start()

    def s1_compute(b, d, si):
        base = ebase0 + b * W
        pltpu.make_async_copy(qtab.at[gq.at[si]], qbuf.at[si], sq[si]).wait()
        pltpu.make_async_copy(ktab.at[gk.at[si]], kbuf.at[si], sk[si]).wait()

        @plsc.parallel_loop(0, W, unroll=4)
        def _(e):
            a = jnp.zeros((LN,), F32)
            for j in range(CH // (2 * LN)):
                sl = pl.ds(j * 2 * LN, 2 * LN)
                qa, qb = plsc.unpack(qbuf[si, e, sl],
                                     format=plsc.PackFormat.INTERLEAVED,
                                     preferred_element_type=F32)
                ka, kb = plsc.unpack(kbuf[si, e, sl],
                                     format=plsc.PackFormat.INTERLEAVED,
                                     preferred_element_type=F32)
                a = a + qa * ka + qb * kb
            plsc.store_scatter(exb.at[si], [jnp.full((LN,), e, I32)],
                               plsc.cumsum(a), mask=m_last)

        for j in range(W // LN):
            sl = pl.ds(j * LN, LN)
            exb[si, sl] = jnp.exp(exb[si, sl] * (1.0 / 16.0))
        pltpu.sync_copy(exb.at[si], ex_out.at[pl.ds(cep + base, W)])

        @plsc.parallel_loop(0, W, unroll=4)
        def _(e):
            wv = plsc.load_gather(exb.at[si], [jnp.full((LN,), e, I32)])
            vbuf[si, e, pl.ds(0, LN)] = wv

        pltpu.make_async_copy(vbuf.at[si], acc.at[dstb.at[d]],
                              sa[si]).start(add=True)

    zero_acc()
    plsc.subcore_barrier()

    def scat_wait(d, si):
        pltpu.make_async_copy(vbuf.at[si], acc.at[dstb.at[d]], sa[si]).wait()

    for u in range(3):
        s1_raw(u, u)
    s1_gather(0, 0, 0)

    @pl.loop(0, nb, step=4)
    def _(b):
        for u in range(4):
            if u == 0:
                @pl.when(b >= 1)
                def _():
                    scat_wait(3, 1)
            else:
                scat_wait(u - 1, (u - 1) % 2)

            @pl.when(b + u + 1 < nb)
            def _():
                s1_gather(b + u + 1, (u + 1) % 4, (u + 1) % 2)

            s1_compute(b + u, u, u % 2)

            @pl.when(b + u + 3 < nb)
            def _():
                s1_raw(b + u + 3, (u + 3) % 4)

    scat_wait((nb - 1) % 4, (nb - 1) % 2)
    plsc.subcore_barrier()
    pltpu.sync_copy(acc.at[pl.ds(t * rpt, rpt)],
                    den_out.at[pl.ds(cnp + t * rpt, rpt)])

    for half in range(NS):
        voff = (NS * c + half) * np_

        zero_acc()
        plsc.subcore_barrier()

        def sw_raw(b, d):
            base = ebase0 + b * W
            pltpu.make_async_copy(dst_hbm.at[pl.ds(base, W)], dstb.at[d],
                                  sr[d]).start()
            pltpu.make_async_copy(src_hbm.at[pl.ds(base, W)], srcb.at[d],
                                  sr[d]).start()

        def sw_gather(b, d, si):
            base = ebase0 + b * W
            pltpu.make_async_copy(dst_hbm.at[pl.ds(base, W)], dstb.at[d],
                                  sr[d]).wait()
            pltpu.make_async_copy(src_hbm.at[pl.ds(base, W)], srcb.at[d],
                                  sr[d]).wait()
            for j in range(W // LN):
                sl = pl.ds(j * LN, LN)
                gq[si, sl] = srcb[d, sl] + voff
            pltpu.make_async_copy(vtab.at[gq.at[si]], vbuf.at[si],
                                  sq[si]).start()
            pltpu.make_async_copy(ex_out.at[pl.ds(cep + base, W)], exb.at[si],
                                  se[si]).start()

        def sw_compute(b, d, si):
            base = ebase0 + b * W
            pltpu.make_async_copy(vtab.at[gq.at[si]], vbuf.at[si],
                                  sq[si]).wait()
            pltpu.make_async_copy(ex_out.at[pl.ds(cep + base, W)], exb.at[si],
                                  se[si]).wait()

            @plsc.parallel_loop(0, W, unroll=4)
            def _(e):
                wv = plsc.load_gather(exb.at[si], [jnp.full((LN,), e, I32)])
                for j in range(SW // LN):
                    sl = pl.ds(j * LN, LN)
                    vbuf[si, e, sl] = vbuf[si, e, sl] * wv

            pltpu.make_async_copy(vbuf.at[si], acc.at[dstb.at[d]],
                                  sa[si]).start(add=True)

        for u in range(3):
            sw_raw(u, u)
        sw_gather(0, 0, 0)

        @pl.loop(0, nb, step=4)
        def _(b):
            for u in range(4):
                if u == 0:
                    @pl.when(b >= 1)
                    def _():
                        scat_wait(3, 1)
                else:
                    scat_wait(u - 1, (u - 1) % 2)

                @pl.when(b + u + 1 < nb)
                def _():
                    sw_gather(b + u + 1, (u + 1) % 4, (u + 1) % 2)

                sw_compute(b + u, u, u % 2)

                @pl.when(b + u + 3 < nb)
                def _():
                    sw_raw(b + u + 3, (u + 3) % 4)

        scat_wait((nb - 1) % 4, (nb - 1) % 2)
        plsc.subcore_barrier()
        pltpu.sync_copy(acc.at[pl.ds(t * rpt, rpt)],
                        msg_out.at[pl.ds(voff + t * rpt, rpt)])
        plsc.subcore_barrier()


@functools.lru_cache(maxsize=None)
def _sc_call(np_, s_per, nb):
    mesh = plsc.VectorSubcoreMesh(core_axis_name="c", subcore_axis_name="s")
    cp = pltpu.CompilerParams()
    for fld, val in (("needs_layout_passes", False),
                     ("use_tc_tiling_on_sc", False)):
        if fld in pltpu.CompilerParams.__dataclass_fields__:
            cp = dataclasses.replace(cp, **{fld: val})
    return pl.kernel(
        functools.partial(_sc_body, np_=np_, s_per=s_per, nb=nb),
        out_type=(
            jax.ShapeDtypeStruct((H * np_, SW), F32),
            jax.ShapeDtypeStruct((H * NS * np_, SW), F32),
            jax.ShapeDtypeStruct((H * NSUB * s_per,), F32),
        ),
        mesh=mesh,
        compiler_params=cp,
        scratch_types=[
            pltpu.VMEM((4, W), I32),
            pltpu.VMEM((4, W), I32),
            pltpu.VMEM((2, W), I32),
            pltpu.VMEM((2, W), I32),
            pltpu.VMEM((2, W, CH), BF16),
            pltpu.VMEM((2, W, CH), BF16),
            pltpu.VMEM((2, W, SW), F32),
            pltpu.VMEM((2, W), F32),
            pltpu.SemaphoreType.DMA,
            pltpu.SemaphoreType.DMA,
            pltpu.SemaphoreType.DMA,
            pltpu.SemaphoreType.DMA,
            pltpu.SemaphoreType.DMA,
            pltpu.SemaphoreType.DMA,
            pltpu.SemaphoreType.DMA,
            pltpu.SemaphoreType.DMA,
            pltpu.SemaphoreType.DMA,
            pltpu.SemaphoreType.DMA,
            pltpu.SemaphoreType.DMA,
            pltpu.SemaphoreType.DMA,
            pltpu.VMEM_SHARED((np_, SW), F32),
        ],
    )


def _fin_body(msg_ref, den_ref, s_ref, o_ref, *, act):
    o = s_ref[...]
    ms = []
    for h in range(H):
        d = den_ref[h, :, 0:1] + 1e-16
        m = jnp.concatenate([msg_ref[h, j] for j in range(NS)], axis=1) / d
        ms.append(m)
    o = o + sum(ms) * (1.0 / H)
    if act:
        o = jnp.where(o >= 0, o, 0.1 * o)
    o_ref[...] = o


@functools.lru_cache(maxsize=None)
def _fin_call(np_, bn, act):
    return pl.pallas_call(
        functools.partial(_fin_body, act=act),
        grid=(np_ // bn,),
        in_specs=[
            pl.BlockSpec((H, NS, bn, SW), lambda i: (0, 0, i, 0)),
            pl.BlockSpec((H, bn, SW), lambda i: (0, i, 0)),
            pl.BlockSpec((bn, CH), lambda i: (i, 0)),
        ],
        out_specs=pl.BlockSpec((bn, CH), lambda i: (i, 0)),
        out_shape=jax.ShapeDtypeStruct((np_, CH), F32),
    )


def _layer(xp, dstp, srcp, zin, wq, bq, wk, bk, wv, bv, ws, bs,
           np_, s_per, nb, n_valid, act):
    q, k, v, s = _qkv_call(np_, xp.shape[1], n_valid, BN)(
        xp, wq, bq.reshape(1, -1), wk, bk.reshape(1, -1),
        wv, bv.reshape(1, -1), ws, bs.reshape(1, -1))
    den, msg, _ = _sc_call(np_, s_per, nb)(
        q.reshape(H * np_, CH), k.reshape(H * np_, CH),
        v.reshape(H * NS * np_, SW), dstp, srcp, zin)
    return _fin_call(np_, BN, act)(
        msg.reshape(H, NS, np_, SW), den.reshape(H, np_, SW), s)


def kernel(x, edge_index_list, Wq1, bq1, Wk1, bk1, Wv1, bv1, Ws1, bs1,
           Wq2, bq2, Wk2, bk2, Wv2, bv2, Ws2, bs2):
    B, N, Cin = x.shape
    E = B * edge_index_list.shape[2]
    offs = (jnp.arange(B, dtype=edge_index_list.dtype) * N)[:, None, None]
    flat = jnp.transpose(edge_index_list + offs, (1, 0, 2)).reshape(2, -1)
    src = flat[0].astype(I32)
    dst = flat[1].astype(I32)

    np_ = ((B * N + BN) // BN) * BN
    s_per = ((E + NSUB * 4 * W - 1) // (NSUB * 4 * W)) * 4 * W
    nb = s_per // W
    ep = NSUB * s_per
    pad = jnp.full((ep - E,), B * N, I32)
    srcp = jnp.concatenate([src, pad])
    dstp = jnp.concatenate([dst, pad])
    xp = jnp.pad(x.reshape(B * N, Cin), ((0, np_ - B * N), (0, 0)))
    zin = jnp.zeros((np_ // NSUB, SW), F32)

    h = _layer(xp, dstp, srcp, zin, Wq1, bq1, Wk1, bk1, Wv1, bv1, Ws1, bs1,
               np_, s_per, nb, B * N, act=True)
    o = _layer(h, dstp, srcp, zin, Wq2, bq2, Wk2, bk2, Wv2, bv2, Ws2, bs2,
               np_, s_per, nb, B * N, act=False)
    return o[:B * N].reshape(B, N, CH)


# speedup vs baseline: 19.0218x; 1.0172x over previous
import dataclasses
import functools

import jax
import jax.numpy as jnp
from jax import lax
from jax.experimental import pallas as pl
from jax.experimental.pallas import tpu as pltpu
from jax.experimental.pallas import tpu_sc as plsc

F32 = jnp.float32
BF16 = jnp.bfloat16
I32 = jnp.int32
LN = 16
H = 2
CH = 256
SW = 128
NS = CH // SW
W = 64
NSUB = 16
BN = 1264


def _qkv_body(x_ref, wq_ref, bq_ref, wk_ref, bk_ref, wv_ref, bv_ref,
              ws_ref, bs_ref, q_ref, k_ref, v_ref, s_ref, *, bn, n_valid):
    i = pl.program_id(0)
    rows = i * bn + lax.broadcasted_iota(I32, (bn, 1), 0)
    m = (rows < n_valid).astype(F32)
    x = x_ref[...] * m
    q = (jnp.dot(x, wq_ref[...], preferred_element_type=F32) + bq_ref[...]) * m
    k = (jnp.dot(x, wk_ref[...], preferred_element_type=F32) + bk_ref[...]) * m
    v = (jnp.dot(x, wv_ref[...], preferred_element_type=F32) + bv_ref[...]) * m
    s = (jnp.dot(x, ws_ref[...], preferred_element_type=F32) + bs_ref[...]) * m
    q16, k16 = q.astype(BF16), k.astype(BF16)
    q_ref[0], q_ref[1] = q16[:, :CH], q16[:, CH:]
    k_ref[0], k_ref[1] = k16[:, :CH], k16[:, CH:]
    for j in range(H * NS):
        v_ref[j] = v[:, j * SW:(j + 1) * SW]
    s_ref[...] = s


@functools.lru_cache(maxsize=None)
def _qkv_call(np_, cin, n_valid, bn):
    return pl.pallas_call(
        functools.partial(_qkv_body, bn=bn, n_valid=n_valid),
        grid=(np_ // bn,),
        in_specs=[
            pl.BlockSpec((bn, cin), lambda i: (i, 0)),
            pl.BlockSpec((cin, H * CH), lambda i: (0, 0)),
            pl.BlockSpec((1, H * CH), lambda i: (0, 0)),
            pl.BlockSpec((cin, H * CH), lambda i: (0, 0)),
            pl.BlockSpec((1, H * CH), lambda i: (0, 0)),
            pl.BlockSpec((cin, H * CH), lambda i: (0, 0)),
            pl.BlockSpec((1, H * CH), lambda i: (0, 0)),
            pl.BlockSpec((cin, CH), lambda i: (0, 0)),
            pl.BlockSpec((1, CH), lambda i: (0, 0)),
        ],
        out_specs=[
            pl.BlockSpec((H, bn, CH), lambda i: (0, i, 0)),
            pl.BlockSpec((H, bn, CH), lambda i: (0, i, 0)),
            pl.BlockSpec((H * NS, bn, SW), lambda i: (0, i, 0)),
            pl.BlockSpec((bn, CH), lambda i: (i, 0)),
        ],
        out_shape=[
            jax.ShapeDtypeStruct((H, np_, CH), BF16),
            jax.ShapeDtypeStruct((H, np_, CH), BF16),
            jax.ShapeDtypeStruct((H * NS, np_, SW), F32),
            jax.ShapeDtypeStruct((np_, CH), F32),
        ],
    )


def _sc_body(qtab, ktab, vtab, dst_hbm, src_hbm, zin, den_out, msg_out, ex_out,
             dstb, srcb, gq, gk, qbuf, kbuf, vbuf, exb,
             sr0, sr1, sr2, sr3, sq0, sq1, sk0, sk1, se0, se1, sa0, sa1,
             acc, *, np_, s_per, nb):
    c = lax.axis_index("c")
    t = lax.axis_index("s")
    ep = NSUB * s_per
    rpt = np_ // NSUB
    cnp = c * np_
    cep = c * ep
    lane = lax.iota(I32, LN)
    m_last = lane == (LN - 1)
    sr = (sr0, sr1, sr2, sr3)
    sq = (sq0, sq1)
    sk = (sk0, sk1)
    se = (se0, se1)
    sa = (sa0, sa1)
    ebase0 = t * s_per

    def zero_acc():
        pltpu.sync_copy(zin, acc.at[pl.ds(t * rpt, rpt)])

    def s1_raw(b, d):
        base = ebase0 + b * W
        pltpu.make_async_copy(dst_hbm.at[pl.ds(base, W)], dstb.at[d],
                              sr[d]).start()
        pltpu.make_async_copy(src_hbm.at[pl.ds(base, W)], srcb.at[d],
                              sr[d]).start()

    def s1_gather(b, d, si):
        base = ebase0 + b * W
        pltpu.make_async_copy(dst_hbm.at[pl.ds(base, W)], dstb.at[d],
                              sr[d]).wait()
        pltpu.make_async_copy(src_hbm.at[pl.ds(base, W)], srcb.at[d],
                              sr[d]).wait()
        for j in range(W // LN):
            sl = pl.ds(j * LN, LN)
            gq[si, sl] = dstb[d, sl] + cnp
            gk[si, sl] = srcb[d, sl] + cnp
        pltpu.make_async_copy(qtab.at[gq.at[si]], qbuf.at[si], sq[si]).start()
        pltpu.make_async_copy(ktab.at[gk.at[si]], kbuf.at[si], sk[si]).start()

    def exw_wait(bprev, si):
        pltpu.make_async_copy(exb.at[si],
                              ex_out.at[pl.ds(cep + ebase0 + bprev * W, W)],
                              se[si]).wait()

    def s1_compute(b, d, si):
        base = ebase0 + b * W
        pltpu.make_async_copy(qtab.at[gq.at[si]], qbuf.at[si], sq[si]).wait()
        pltpu.make_async_copy(ktab.at[gk.at[si]], kbuf.at[si], sk[si]).wait()

        @plsc.parallel_loop(0, W, unroll=4)
        def _(e):
            a = jnp.zeros((LN,), F32)
            for j in range(CH // (2 * LN)):
                sl = pl.ds(j * 2 * LN, 2 * LN)
                p = qbuf[si, e, sl] * kbuf[si, e, sl]
                pa, pb = plsc.unpack(p, format=plsc.PackFormat.INTERLEAVED,
                                     preferred_element_type=F32)
                a = a + pa + pb
            plsc.store_scatter(exb.at[si], [jnp.full((LN,), e, I32)],
                               plsc.cumsum(a), mask=m_last)

        for j in range(W // LN):
            sl = pl.ds(j * LN, LN)
            exb[si, sl] = jnp.exp(exb[si, sl] * (1.0 / 16.0))
        pltpu.make_async_copy(exb.at[si], ex_out.at[pl.ds(cep + base, W)],
                              se[si]).start()

        @plsc.parallel_loop(0, W, unroll=4)
        def _(e):
            wv = plsc.load_gather(exb.at[si], [jnp.full((LN,), e, I32)])
            vbuf[si, e, pl.ds(0, LN)] = wv

        pltpu.make_async_copy(vbuf.at[si], acc.at[dstb.at[d]],
                              sa[si]).start(add=True)

    zero_acc()
    plsc.subcore_barrier()

    def scat_wait(d, si):
        pltpu.make_async_copy(vbuf.at[si], acc.at[dstb.at[d]], sa[si]).wait()

    for u in range(3):
        s1_raw(u, u)
    s1_gather(0, 0, 0)

    @pl.loop(0, nb, step=4)
    def _(b):
        for u in range(4):
            if u == 0:
                @pl.when(b >= 1)
                def _():
                    scat_wait(3, 1)
            else:
                scat_wait(u - 1, (u - 1) % 2)

            @pl.when(b + u + 1 < nb)
            def _():
                s1_gather(b + u + 1, (u + 1) % 4, (u + 1) % 2)

            if u < 2:
                @pl.when(b + u >= 2)
                def _():
                    exw_wait(b + u - 2, u % 2)
            else:
                exw_wait(b + u - 2, u % 2)

            s1_compute(b + u, u, u % 2)

            @pl.when(b + u + 3 < nb)
            def _():
                s1_raw(b + u + 3, (u + 3) % 4)

    scat_wait((nb - 1) % 4, (nb - 1) % 2)
    exw_wait(nb - 2, 0)
    exw_wait(nb - 1, 1)
    plsc.subcore_barrier()
    pltpu.sync_copy(acc.at[pl.ds(t * rpt, rpt)],
                    den_out.at[pl.ds(cnp + t * rpt, rpt)])

    for half in range(NS):
        voff = (NS * c + half) * np_

        zero_acc()
        plsc.subcore_barrier()

        def sw_raw(b, d):
            base = ebase0 + b * W
            pltpu.make_async_copy(dst_hbm.at[pl.ds(base, W)], dstb.at[d],
                                  sr[d]).start()
            pltpu.make_async_copy(src_hbm.at[pl.ds(base, W)], srcb.at[d],
                                  sr[d]).start()

        def sw_gather(b, d, si):
            base = ebase0 + b * W
            pltpu.make_async_copy(dst_hbm.at[pl.ds(base, W)], dstb.at[d],
                                  sr[d]).wait()
            pltpu.make_async_copy(src_hbm.at[pl.ds(base, W)], srcb.at[d],
                                  sr[d]).wait()
            for j in range(W // LN):
                sl = pl.ds(j * LN, LN)
                gq[si, sl] = srcb[d, sl] + voff
            pltpu.make_async_copy(vtab.at[gq.at[si]], vbuf.at[si],
                                  sq[si]).start()
            pltpu.make_async_copy(ex_out.at[pl.ds(cep + base, W)], exb.at[si],
                                  se[si]).start()

        def sw_compute(b, d, si):
            base = ebase0 + b * W
            pltpu.make_async_copy(vtab.at[gq.at[si]], vbuf.at[si],
                                  sq[si]).wait()
            pltpu.make_async_copy(ex_out.at[pl.ds(cep + base, W)], exb.at[si],
                                  se[si]).wait()

            @plsc.parallel_loop(0, W, unroll=4)
            def _(e):
                wv = plsc.load_gather(exb.at[si], [jnp.full((LN,), e, I32)])
                for j in range(SW // LN):
                    sl = pl.ds(j * LN, LN)
                    vbuf[si, e, sl] = vbuf[si, e, sl] * wv

            pltpu.make_async_copy(vbuf.at[si], acc.at[dstb.at[d]],
                                  sa[si]).start(add=True)

        for u in range(3):
            sw_raw(u, u)
        sw_gather(0, 0, 0)

        @pl.loop(0, nb, step=4)
        def _(b):
            for u in range(4):
                if u == 0:
                    @pl.when(b >= 1)
                    def _():
                        scat_wait(3, 1)
                else:
                    scat_wait(u - 1, (u - 1) % 2)

                @pl.when(b + u + 1 < nb)
                def _():
                    sw_gather(b + u + 1, (u + 1) % 4, (u + 1) % 2)

                sw_compute(b + u, u, u % 2)

                @pl.when(b + u + 3 < nb)
                def _():
                    sw_raw(b + u + 3, (u + 3) % 4)

        scat_wait((nb - 1) % 4, (nb - 1) % 2)
        plsc.subcore_barrier()
        pltpu.sync_copy(acc.at[pl.ds(t * rpt, rpt)],
                        msg_out.at[pl.ds(voff + t * rpt, rpt)])
        plsc.subcore_barrier()


@functools.lru_cache(maxsize=None)
def _sc_call(np_, s_per, nb):
    mesh = plsc.VectorSubcoreMesh(core_axis_name="c", subcore_axis_name="s")
    cp = pltpu.CompilerParams()
    for fld, val in (("needs_layout_passes", False),
                     ("use_tc_tiling_on_sc", False)):
        if fld in pltpu.CompilerParams.__dataclass_fields__:
            cp = dataclasses.replace(cp, **{fld: val})
    return pl.kernel(
        functools.partial(_sc_body, np_=np_, s_per=s_per, nb=nb),
        out_type=(
            jax.ShapeDtypeStruct((H * np_, SW), F32),
            jax.ShapeDtypeStruct((H * NS * np_, SW), F32),
            jax.ShapeDtypeStruct((H * NSUB * s_per,), F32),
        ),
        mesh=mesh,
        compiler_params=cp,
        scratch_types=[
            pltpu.VMEM((4, W), I32),
            pltpu.VMEM((4, W), I32),
            pltpu.VMEM((2, W), I32),
            pltpu.VMEM((2, W), I32),
            pltpu.VMEM((2, W, CH), BF16),
            pltpu.VMEM((2, W, CH), BF16),
            pltpu.VMEM((2, W, SW), F32),
            pltpu.VMEM((2, W), F32),
            pltpu.SemaphoreType.DMA,
            pltpu.SemaphoreType.DMA,
            pltpu.SemaphoreType.DMA,
            pltpu.SemaphoreType.DMA,
            pltpu.SemaphoreType.DMA,
            pltpu.SemaphoreType.DMA,
            pltpu.SemaphoreType.DMA,
            pltpu.SemaphoreType.DMA,
            pltpu.SemaphoreType.DMA,
            pltpu.SemaphoreType.DMA,
            pltpu.SemaphoreType.DMA,
            pltpu.SemaphoreType.DMA,
            pltpu.VMEM_SHARED((np_, SW), F32),
        ],
    )


def _fin_body(msg_ref, den_ref, s_ref, o_ref, *, act):
    o = s_ref[...]
    ms = []
    for h in range(H):
        d = den_ref[h, :, 0:1] + 1e-16
        m = jnp.concatenate([msg_ref[h, j] for j in range(NS)], axis=1) / d
        ms.append(m)
    o = o + sum(ms) * (1.0 / H)
    if act:
        o = jnp.where(o >= 0, o, 0.1 * o)
    o_ref[...] = o


@functools.lru_cache(maxsize=None)
def _fin_call(np_, bn, act):
    return pl.pallas_call(
        functools.partial(_fin_body, act=act),
        grid=(np_ // bn,),
        in_specs=[
            pl.BlockSpec((H, NS, bn, SW), lambda i: (0, 0, i, 0)),
            pl.BlockSpec((H, bn, SW), lambda i: (0, i, 0)),
            pl.BlockSpec((bn, CH), lambda i: (i, 0)),
        ],
        out_specs=pl.BlockSpec((bn, CH), lambda i: (i, 0)),
        out_shape=jax.ShapeDtypeStruct((np_, CH), F32),
    )


def _layer(xp, dstp, srcp, zin, wq, bq, wk, bk, wv, bv, ws, bs,
           np_, s_per, nb, n_valid, act):
    q, k, v, s = _qkv_call(np_, xp.shape[1], n_valid, BN)(
        xp, wq, bq.reshape(1, -1), wk, bk.reshape(1, -1),
        wv, bv.reshape(1, -1), ws, bs.reshape(1, -1))
    den, msg, _ = _sc_call(np_, s_per, nb)(
        q.reshape(H * np_, CH), k.reshape(H * np_, CH),
        v.reshape(H * NS * np_, SW), dstp, srcp, zin)
    return _fin_call(np_, BN, act)(
        msg.reshape(H, NS, np_, SW), den.reshape(H, np_, SW), s)


def kernel(x, edge_index_list, Wq1, bq1, Wk1, bk1, Wv1, bv1, Ws1, bs1,
           Wq2, bq2, Wk2, bk2, Wv2, bv2, Ws2, bs2):
    B, N, Cin = x.shape
    E = B * edge_index_list.shape[2]
    offs = (jnp.arange(B, dtype=edge_index_list.dtype) * N)[:, None, None]
    flat = jnp.transpose(edge_index_list + offs, (1, 0, 2)).reshape(2, -1)
    src = flat[0].astype(I32)
    dst = flat[1].astype(I32)

    np_ = ((B * N + BN) // BN) * BN
    s_per = ((E + NSUB * 4 * W - 1) // (NSUB * 4 * W)) * 4 * W
    nb = s_per // W
    ep = NSUB * s_per
    pad = jnp.full((ep - E,), B * N, I32)
    srcp = jnp.concatenate([src, pad])
    dstp = jnp.concatenate([dst, pad])
    xp = jnp.pad(x.reshape(B * N, Cin), ((0, np_ - B * N), (0, 0)))
    zin = jnp.zeros((np_ // NSUB, SW), F32)

    h = _layer(xp, dstp, srcp, zin, Wq1, bq1, Wk1, bk1, Wv1, bv1, Ws1, bs1,
               np_, s_per, nb, B * N, act=True)
    o = _layer(h, dstp, srcp, zin, Wq2, bq2, Wk2, bk2, Wv2, bv2, Ws2, bs2,
               np_, s_per, nb, B * N, act=False)
    return o[:B * N].reshape(B, N, CH)
